# Initial kernel scaffold; baseline (speedup 1.0000x reference)
import functools

import jax
import jax.numpy as jnp
from jax import lax
from jax.experimental import pallas as pl
from jax.experimental.pallas import tpu as pltpu
from jax.experimental.pallas import tpu_sc as plsc

N = 10000
NP = 10240
G = 128
NCLS = 40
DIN = 128
HID = 128
H2 = 256
EPS = 1e-5

CH = 128
TILES = 16
STRIPE = NP // TILES
E_TOT = 320000 + N
T_CH = 162
E_PAD = TILES * T_CH * CH
DEG_CH = T_CH // 2

BM = 512
NB = NP // BM

_mesh = plsc.VectorSubcoreMesh(core_axis_name="c", subcore_axis_name="s")


def _fill2d(ref, rows, cols, value):
    vec = jnp.full((16,), value, jnp.float32)

    def body(r, _):
        for c in range(cols // 16):
            ref[r, pl.ds(c * 16, 16)] = vec
        return 0

    lax.fori_loop(0, rows, body, 0)


def _deg_body(dstr, deg_out, idx_v, ones_v, deg_sp):
    cid = lax.axis_index("c")
    wid = lax.axis_index("s")
    row0 = wid * STRIPE
    _fill2d(ones_v, CH, 16, 0.0)
    for k in range(STRIPE // CH):
        pltpu.sync_copy(ones_v, deg_sp.at[pl.ds(row0 + k * CH, CH)])
    _fill2d(ones_v, CH, 16, 1.0)
    base = (cid * TILES + wid) * DEG_CH
    pltpu.sync_copy(dstr.at[pl.ds(base, DEG_CH)], idx_v)
    plsc.subcore_barrier()

    def step(j, _):
        pltpu.sync_copy(ones_v, deg_sp.at[idx_v.at[j]], add=True)
        return 0

    lax.fori_loop(0, DEG_CH, step, 0)
    plsc.subcore_barrier()
    pltpu.sync_copy(deg_sp.at[pl.ds(row0, STRIPE)],
                    deg_out.at[cid, pl.ds(row0, STRIPE)])


def _deg_call(dstr):
    return pl.kernel(
        _deg_body,
        out_type=jax.ShapeDtypeStruct((2, NP, 16), jnp.float32),
        mesh=_mesh,
        scratch_types=[
            pltpu.VMEM((DEG_CH, CH), jnp.int32),
            pltpu.VMEM((CH, 16), jnp.float32),
            pltpu.VMEM_SHARED((NP, 16), jnp.float32),
        ],
    )(dstr)


def _agg_body(Dh, h0, h1, srcr, dstr, out, sidx, didx, rows0, rows1,
              acc_sp, sem0, sem1):
    cid = lax.axis_index("c")
    wid = lax.axis_index("s")
    row0 = wid * STRIPE
    _fill2d(rows0, CH, Dh, 0.0)
    for k in range(STRIPE // CH):
        pltpu.sync_copy(rows0, acc_sp.at[pl.ds(row0 + k * CH, CH)])
    pltpu.sync_copy(srcr.at[pl.ds(wid * T_CH, T_CH)], sidx)
    pltpu.sync_copy(dstr.at[pl.ds(wid * T_CH, T_CH)], didx)
    plsc.subcore_barrier()

    def run(h):
        pltpu.async_copy(h.at[sidx.at[0]], rows0, sem0)

        def step(i, _):
            j0 = 2 * i
            j1 = 2 * i + 1
            jn = jnp.minimum(j1 + 1, T_CH - 1)
            pltpu.make_async_copy(h.at[sidx.at[j0]], rows0, sem0).wait()
            pltpu.async_copy(h.at[sidx.at[j1]], rows1, sem1)
            pltpu.sync_copy(rows0, acc_sp.at[didx.at[j0]], add=True)
            pltpu.make_async_copy(h.at[sidx.at[j1]], rows1, sem1).wait()
            pltpu.async_copy(h.at[sidx.at[jn]], rows0, sem0)
            pltpu.sync_copy(rows1, acc_sp.at[didx.at[j1]], add=True)
            return 0

        lax.fori_loop(0, T_CH // 2, step, 0)
        pltpu.make_async_copy(h.at[sidx.at[T_CH - 1]], rows0, sem0).wait()

    pl.when(cid == 0)(lambda: run(h0))
    pl.when(cid == 1)(lambda: run(h1))
    plsc.subcore_barrier()
    pltpu.sync_copy(acc_sp.at[pl.ds(row0, STRIPE)],
                    out.at[cid, pl.ds(row0, STRIPE)])


def _agg_call(Dh, h0, h1, srcr, dstr):
    return pl.kernel(
        functools.partial(_agg_body, Dh),
        out_type=jax.ShapeDtypeStruct((2, NP, Dh), jnp.float32),
        mesh=_mesh,
        scratch_types=[
            pltpu.VMEM((T_CH, CH), jnp.int32),
            pltpu.VMEM((T_CH, CH), jnp.int32),
            pltpu.VMEM((CH, Dh), jnp.float32),
            pltpu.VMEM((CH, Dh), jnp.float32),
            pltpu.VMEM_SHARED((NP, Dh), jnp.float32),
            pltpu.SemaphoreType.DMA,
            pltpu.SemaphoreType.DMA,
        ],
    )(h0, h1, srcr, dstr)


def _lin1_body(x_ref, w_ref, dega_ref, degb_ref, ha_ref, hb_ref, dinv_ref):
    deg = dega_ref[0][:, 0:1] + degb_ref[0][:, 0:1]
    dinv = jnp.where(deg > 0, lax.rsqrt(deg), 0.0)
    h = jnp.dot(x_ref[...], w_ref[...], preferred_element_type=jnp.float32)
    hs = h * dinv
    ha_ref[...] = hs[:, :HID]
    hb_ref[...] = hs[:, HID:]
    dinv_ref[...] = dinv


def _lin1_call(xp, W1, deg):
    return pl.pallas_call(
        _lin1_body,
        grid=(NB,),
        in_specs=[
            pl.BlockSpec((BM, DIN), lambda i: (i, 0)),
            pl.BlockSpec((DIN, H2), lambda i: (0, 0)),
            pl.BlockSpec((1, BM, 16), lambda i: (0, i, 0)),
            pl.BlockSpec((1, BM, 16), lambda i: (1, i, 0)),
        ],
        out_specs=[
            pl.BlockSpec((BM, HID), lambda i: (i, 0)),
            pl.BlockSpec((BM, HID), lambda i: (i, 0)),
            pl.BlockSpec((BM, 1), lambda i: (i, 0)),
        ],
        out_shape=[
            jax.ShapeDtypeStruct((NP, HID), jnp.float32),
            jax.ShapeDtypeStruct((NP, HID), jnp.float32),
            jax.ShapeDtypeStruct((NP, 1), jnp.float32),
        ],
    )(xp, W1, deg, deg)


def _mid_body(agga_ref, aggb_ref, dinv_ref, g_ref, b_ref, w2_ref,
              oa_ref, ob_ref, stats):
    ph = pl.program_id(0)
    blk = pl.program_id(1)
    dinv = dinv_ref[...]
    ya = agga_ref[0] * dinv
    yb = aggb_ref[0] * dinv

    @pl.when((ph == 0) & (blk == 0))
    def _():
        stats[...] = jnp.zeros_like(stats)

    @pl.when(ph == 0)
    def _():
        rid = blk * BM + lax.broadcasted_iota(jnp.int32, (BM, HID), 0)
        m = rid < N
        yam = jnp.where(m, ya, 0.0)
        ybm = jnp.where(m, yb, 0.0)
        stats[0:1, :] += jnp.sum(yam, axis=0, keepdims=True)
        stats[1:2, :] += jnp.sum(ybm, axis=0, keepdims=True)
        stats[2:3, :] += jnp.sum(yam * yam, axis=0, keepdims=True)
        stats[3:4, :] += jnp.sum(ybm * ybm, axis=0, keepdims=True)

    @pl.when(ph == 1)
    def _():
        inv_n = jnp.float32(1.0 / N)
        ma = stats[0:1, :] * inv_n
        mb = stats[1:2, :] * inv_n
        va = stats[2:3, :] * inv_n - ma * ma
        vb = stats[3:4, :] * inv_n - mb * mb
        sa = lax.rsqrt(va + EPS) * g_ref[0:1, :]
        sb = lax.rsqrt(vb + EPS) * g_ref[1:2, :]
        za = jnp.maximum((ya - ma) * sa + b_ref[0:1, :], 0.0)
        zb = jnp.maximum((yb - mb) * sb + b_ref[1:2, :], 0.0)
        t = (jnp.dot(za, w2_ref[:HID, :], preferred_element_type=jnp.float32)
             + jnp.dot(zb, w2_ref[HID:, :],
                       preferred_element_type=jnp.float32))
        ts = t * dinv
        oa_ref[...] = ts[:, :HID // 2]
        ob_ref[...] = ts[:, HID // 2:]


def _mid_call(agg1, dinv, g1, b1r, W2):
    return pl.pallas_call(
        _mid_body,
        grid=(2, NB),
        in_specs=[
            pl.BlockSpec((1, BM, HID), lambda p, i: (0, i, 0)),
            pl.BlockSpec((1, BM, HID), lambda p, i: (1, i, 0)),
            pl.BlockSpec((BM, 1), lambda p, i: (i, 0)),
            pl.BlockSpec((2, HID), lambda p, i: (0, 0)),
            pl.BlockSpec((2, HID), lambda p, i: (0, 0)),
            pl.BlockSpec((H2, HID), lambda p, i: (0, 0)),
        ],
        out_specs=[
            pl.BlockSpec((BM, HID // 2), lambda p, i: (i, 0)),
            pl.BlockSpec((BM, HID // 2), lambda p, i: (i, 0)),
        ],
        out_shape=[
            jax.ShapeDtypeStruct((NP, HID // 2), jnp.float32),
            jax.ShapeDtypeStruct((NP, HID // 2), jnp.float32),
        ],
        scratch_shapes=[pltpu.VMEM((4, HID), jnp.float32)],
    )(agg1, agg1, dinv, g1, b1r, W2)


def _final_body(ea_ref, eb_ref, dinv_ref, g_ref, b_ref, batch_ref,
                wn_ref, bn_ref, wf_ref, bf_ref, out_ref, stats, pooled):
    ph = pl.program_id(0)
    blk = pl.program_id(1)
    dinv = dinv_ref[...]
    y = jnp.concatenate([ea_ref[0], eb_ref[0]], axis=1) * dinv
    rid = blk * BM + lax.broadcasted_iota(jnp.int32, (BM, HID), 0)
    m = rid < N

    @pl.when((ph == 0) & (blk == 0))
    def _():
        stats[...] = jnp.zeros_like(stats)
        pooled[...] = jnp.zeros_like(pooled)

    @pl.when(ph == 0)
    def _():
        ym = jnp.where(m, y, 0.0)
        stats[0:1, :] += jnp.sum(ym, axis=0, keepdims=True)
        stats[1:2, :] += jnp.sum(ym * ym, axis=0, keepdims=True)

    @pl.when(ph == 1)
    def _():
        inv_n = jnp.float32(1.0 / N)
        mean = stats[0:1, :] * inv_n
        var = stats[1:2, :] * inv_n - mean * mean
        z = jnp.maximum((y - mean) * lax.rsqrt(var + EPS) * g_ref[...]
                        + b_ref[...], 0.0)
        zm = jnp.where(m, z, 0.0)
        oh = (batch_ref[...] ==
              lax.broadcasted_iota(jnp.int32, (BM, G), 1)).astype(jnp.float32)
        pooled[...] += lax.dot_general(
            oh, zm, (((0,), (0,)), ((), ())),
            preferred_element_type=jnp.float32)

    @pl.when((ph == 1) & (blk == NB - 1))
    def _():
        t = jnp.dot(pooled[...], wn_ref[...],
                    preferred_element_type=jnp.float32) + bn_ref[...]
        lg = jnp.dot(t, wf_ref[...],
                     preferred_element_type=jnp.float32) + bf_ref[...]
        mx = jnp.max(lg, axis=1, keepdims=True)
        lse = jnp.log(jnp.sum(jnp.exp(lg - mx), axis=1, keepdims=True))
        out_ref[...] = lg - mx - lse


def _final_call(agg2, dinv, g2, b2r, batchp, Wn, bnr, Wf, bfr):
    return pl.pallas_call(
        _final_body,
        grid=(2, NB),
        in_specs=[
            pl.BlockSpec((1, BM, HID // 2), lambda p, i: (0, i, 0)),
            pl.BlockSpec((1, BM, HID // 2), lambda p, i: (1, i, 0)),
            pl.BlockSpec((BM, 1), lambda p, i: (i, 0)),
            pl.BlockSpec((1, HID), lambda p, i: (0, 0)),
            pl.BlockSpec((1, HID), lambda p, i: (0, 0)),
            pl.BlockSpec((BM, 1), lambda p, i: (i, 0)),
            pl.BlockSpec((HID, HID), lambda p, i: (0, 0)),
            pl.BlockSpec((1, HID), lambda p, i: (0, 0)),
            pl.BlockSpec((HID, NCLS), lambda p, i: (0, 0)),
            pl.BlockSpec((1, NCLS), lambda p, i: (0, 0)),
        ],
        out_specs=pl.BlockSpec((G, NCLS), lambda p, i: (0, 0)),
        out_shape=jax.ShapeDtypeStruct((G, NCLS), jnp.float32),
        scratch_shapes=[
            pltpu.VMEM((2, HID), jnp.float32),
            pltpu.VMEM((G, HID), jnp.float32),
        ],
    )(agg2, agg2, dinv, g2, b2r, batchp, Wn, bnr, Wf, bfr)


def kernel(x, edge_index, batch, W1, b1, gamma1, beta1, W2, b2, gamma2,
           beta2, Wn, bn, Wf, bf):
    loops = jnp.arange(N, dtype=jnp.int32)
    src = jnp.concatenate([edge_index[0], loops])
    dst = jnp.concatenate([edge_index[1], loops])
    fill = jnp.full((E_PAD - E_TOT,), N, jnp.int32)
    srcr = jnp.concatenate([src, fill]).reshape(E_PAD // CH, CH)
    dstr = jnp.concatenate([dst, fill]).reshape(E_PAD // CH, CH)
    xp = jnp.pad(x, ((0, NP - N), (0, 0)))
    batchp = jnp.pad(batch, (0, NP - N), constant_values=G).reshape(NP, 1)

    deg = _deg_call(dstr)
    h1a, h1b, dinv = _lin1_call(xp, W1, deg)
    agg1 = _agg_call(HID, h1a, h1b, srcr, dstr)
    h2a, h2b = _mid_call(agg1, dinv, gamma1.reshape(2, HID),
                         beta1.reshape(2, HID), W2)
    agg2 = _agg_call(HID // 2, h2a, h2b, srcr, dstr)
    return _final_call(agg2, dinv, gamma2.reshape(1, HID),
                       beta2.reshape(1, HID), batchp, Wn,
                       bn.reshape(1, HID), Wf, bf.reshape(1, NCLS))


# speedup vs baseline: 5.8270x; 5.8270x over previous
import functools

import jax
import jax.numpy as jnp
from jax import lax
from jax.experimental import pallas as pl
from jax.experimental.pallas import tpu as pltpu
from jax.experimental.pallas import tpu_sc as plsc

N = 10000
NP = 10240
G = 128
NCLS = 40
DIN = 128
HID = 128
H2 = 256
EPS = 1e-5

CH = 128
TILES = 16
STRIPE = NP // TILES
E_TOT = 320000 + N
T_CH = 168
E_PAD = TILES * T_CH * CH
DEG_CH = 88
E_PAD_D = 2 * TILES * DEG_CH * CH

BM = 512
NB = NP // BM

_mesh = plsc.VectorSubcoreMesh(core_axis_name="c", subcore_axis_name="s")


def _fill2d(ref, rows, cols, value):
    vec = jnp.full((16,), value, jnp.float32)

    def body(r, _):
        for c in range(cols // 16):
            ref[r, pl.ds(c * 16, 16)] = vec
        return 0

    lax.fori_loop(0, rows, body, 0)


def _deg_body(dstr, deg_out, idx_v, ones_v, deg_sp):
    cid = lax.axis_index("c")
    wid = lax.axis_index("s")
    row0 = wid * STRIPE
    _fill2d(ones_v, CH, 16, 0.0)
    for k in range(STRIPE // CH):
        pltpu.sync_copy(ones_v, deg_sp.at[pl.ds(row0 + k * CH, CH)])
    _fill2d(ones_v, CH, 16, 1.0)
    base = (cid * TILES + wid) * DEG_CH
    pltpu.sync_copy(dstr.at[pl.ds(base, DEG_CH)], idx_v)
    plsc.subcore_barrier()

    def step(j, _):
        pltpu.sync_copy(ones_v, deg_sp.at[idx_v.at[j]], add=True)
        return 0

    lax.fori_loop(0, DEG_CH, step, 0)
    plsc.subcore_barrier()
    pltpu.sync_copy(deg_sp.at[pl.ds(row0, STRIPE)],
                    deg_out.at[cid, pl.ds(row0, STRIPE)])


def _deg_call(dstr):
    return pl.kernel(
        _deg_body,
        out_type=jax.ShapeDtypeStruct((2, NP, 16), jnp.float32),
        mesh=_mesh,
        scratch_types=[
            pltpu.VMEM((DEG_CH, CH), jnp.int32),
            pltpu.VMEM((CH, 16), jnp.float32),
            pltpu.VMEM_SHARED((NP, 16), jnp.float32),
        ],
        compiler_params=pltpu.CompilerParams(use_tc_tiling_on_sc=False),
    )(dstr)


DH = 64


def _agg_body(npass, *args):
    hs = args[:2 * npass]
    (srcr, dstr, out, sidx, didx, rows0, rows1, acc_sp, sem0, sem1) = \
        args[2 * npass:]
    cid = lax.axis_index("c")
    wid = lax.axis_index("s")
    row0 = wid * STRIPE
    pltpu.sync_copy(srcr.at[pl.ds(wid * T_CH, T_CH)], sidx)
    pltpu.sync_copy(dstr.at[pl.ds(wid * T_CH, T_CH)], didx)

    def run(h):
        pltpu.async_copy(h.at[sidx.at[0]], rows0, sem0)

        def step(i, _):
            j0 = 2 * i
            j1 = 2 * i + 1
            jn = jnp.minimum(j1 + 1, T_CH - 1)
            pltpu.make_async_copy(h.at[sidx.at[j0]], rows0, sem0).wait()
            pltpu.async_copy(h.at[sidx.at[j1]], rows1, sem1)
            pltpu.sync_copy(rows0, acc_sp.at[didx.at[j0]], add=True)
            pltpu.make_async_copy(h.at[sidx.at[j1]], rows1, sem1).wait()
            pltpu.async_copy(h.at[sidx.at[jn]], rows0, sem0)
            pltpu.sync_copy(rows1, acc_sp.at[didx.at[j1]], add=True)
            return 0

        lax.fori_loop(0, T_CH // 2, step, 0)
        pltpu.make_async_copy(h.at[sidx.at[T_CH - 1]], rows0, sem0).wait()

    for p in range(npass):
        _fill2d(rows0, CH, DH, 0.0)
        for k in range(STRIPE // CH):
            pltpu.sync_copy(rows0, acc_sp.at[pl.ds(row0 + k * CH, CH)])
        plsc.subcore_barrier()
        pl.when(cid == 0)(functools.partial(run, hs[p]))
        pl.when(cid == 1)(functools.partial(run, hs[npass + p]))
        plsc.subcore_barrier()
        pltpu.sync_copy(
            acc_sp.at[pl.ds(row0, STRIPE)],
            out.at[cid * npass + p, pl.ds(row0, STRIPE)])
        plsc.subcore_barrier()


def _agg_call(npass, hs, srcr, dstr):
    return pl.kernel(
        functools.partial(_agg_body, npass),
        out_type=jax.ShapeDtypeStruct((2 * npass, NP, DH), jnp.float32),
        mesh=_mesh,
        scratch_types=[
            pltpu.VMEM((T_CH, CH), jnp.int32),
            pltpu.VMEM((T_CH, CH), jnp.int32),
            pltpu.VMEM((CH, DH), jnp.float32),
            pltpu.VMEM((CH, DH), jnp.float32),
            pltpu.VMEM_SHARED((NP, DH), jnp.float32),
            pltpu.SemaphoreType.DMA,
            pltpu.SemaphoreType.DMA,
        ],
        compiler_params=pltpu.CompilerParams(use_tc_tiling_on_sc=False),
    )(*hs, srcr, dstr)


def _lin1_body(x_ref, w_ref, dega_ref, degb_ref, h0_ref, h1_ref, h2_ref,
               h3_ref, dinv_ref):
    deg = dega_ref[0][:, 0:1] + degb_ref[0][:, 0:1]
    dinv = jnp.where(deg > 0, lax.rsqrt(deg), 0.0)
    h = jnp.dot(x_ref[...], w_ref[...], preferred_element_type=jnp.float32)
    hs = h * dinv
    h0_ref[...] = hs[:, 0 * DH:1 * DH]
    h1_ref[...] = hs[:, 1 * DH:2 * DH]
    h2_ref[...] = hs[:, 2 * DH:3 * DH]
    h3_ref[...] = hs[:, 3 * DH:4 * DH]
    dinv_ref[...] = dinv


def _lin1_call(xp, W1, deg):
    hspec = pl.BlockSpec((BM, DH), lambda i: (i, 0))
    hshape = jax.ShapeDtypeStruct((NP, DH), jnp.float32)
    return pl.pallas_call(
        _lin1_body,
        grid=(NB,),
        in_specs=[
            pl.BlockSpec((BM, DIN), lambda i: (i, 0)),
            pl.BlockSpec((DIN, H2), lambda i: (0, 0)),
            pl.BlockSpec((1, BM, 16), lambda i: (0, i, 0)),
            pl.BlockSpec((1, BM, 16), lambda i: (1, i, 0)),
        ],
        out_specs=[hspec, hspec, hspec, hspec,
                   pl.BlockSpec((BM, 1), lambda i: (i, 0))],
        out_shape=[hshape, hshape, hshape, hshape,
                   jax.ShapeDtypeStruct((NP, 1), jnp.float32)],
    )(xp, W1, deg, deg)


def _mid_body(a0_ref, a1_ref, a2_ref, a3_ref, dinv_ref, g_ref, b_ref,
              w2_ref, oa_ref, ob_ref, stats):
    ph = pl.program_id(0)
    blk = pl.program_id(1)
    dinv = dinv_ref[...]
    ya = jnp.concatenate([a0_ref[0], a1_ref[0]], axis=1) * dinv
    yb = jnp.concatenate([a2_ref[0], a3_ref[0]], axis=1) * dinv

    @pl.when((ph == 0) & (blk == 0))
    def _():
        stats[...] = jnp.zeros_like(stats)

    @pl.when(ph == 0)
    def _():
        rid = blk * BM + lax.broadcasted_iota(jnp.int32, (BM, HID), 0)
        m = rid < N
        yam = jnp.where(m, ya, 0.0)
        ybm = jnp.where(m, yb, 0.0)
        stats[0:1, :] += jnp.sum(yam, axis=0, keepdims=True)
        stats[1:2, :] += jnp.sum(ybm, axis=0, keepdims=True)
        stats[2:3, :] += jnp.sum(yam * yam, axis=0, keepdims=True)
        stats[3:4, :] += jnp.sum(ybm * ybm, axis=0, keepdims=True)

    @pl.when(ph == 1)
    def _():
        inv_n = jnp.float32(1.0 / N)
        ma = stats[0:1, :] * inv_n
        mb = stats[1:2, :] * inv_n
        va = stats[2:3, :] * inv_n - ma * ma
        vb = stats[3:4, :] * inv_n - mb * mb
        sa = lax.rsqrt(va + EPS) * g_ref[0:1, :]
        sb = lax.rsqrt(vb + EPS) * g_ref[1:2, :]
        za = jnp.maximum((ya - ma) * sa + b_ref[0:1, :], 0.0)
        zb = jnp.maximum((yb - mb) * sb + b_ref[1:2, :], 0.0)
        t = (jnp.dot(za, w2_ref[:HID, :], preferred_element_type=jnp.float32)
             + jnp.dot(zb, w2_ref[HID:, :],
                       preferred_element_type=jnp.float32))
        ts = t * dinv
        oa_ref[...] = ts[:, :HID // 2]
        ob_ref[...] = ts[:, HID // 2:]


def _mid_call(agg1, dinv, g1, b1r, W2):
    return pl.pallas_call(
        _mid_body,
        grid=(2, NB),
        in_specs=[
            pl.BlockSpec((1, BM, DH), lambda p, i: (0, i, 0)),
            pl.BlockSpec((1, BM, DH), lambda p, i: (1, i, 0)),
            pl.BlockSpec((1, BM, DH), lambda p, i: (2, i, 0)),
            pl.BlockSpec((1, BM, DH), lambda p, i: (3, i, 0)),
            pl.BlockSpec((BM, 1), lambda p, i: (i, 0)),
            pl.BlockSpec((2, HID), lambda p, i: (0, 0)),
            pl.BlockSpec((2, HID), lambda p, i: (0, 0)),
            pl.BlockSpec((H2, HID), lambda p, i: (0, 0)),
        ],
        out_specs=[
            pl.BlockSpec((BM, HID // 2), lambda p, i: (i, 0)),
            pl.BlockSpec((BM, HID // 2), lambda p, i: (i, 0)),
        ],
        out_shape=[
            jax.ShapeDtypeStruct((NP, HID // 2), jnp.float32),
            jax.ShapeDtypeStruct((NP, HID // 2), jnp.float32),
        ],
        scratch_shapes=[pltpu.VMEM((4, HID), jnp.float32)],
    )(agg1, agg1, agg1, agg1, dinv, g1, b1r, W2)


def _final_body(ea_ref, eb_ref, dinv_ref, g_ref, b_ref, batch_ref,
                wn_ref, bn_ref, wf_ref, bf_ref, out_ref, stats, pooled):
    ph = pl.program_id(0)
    blk = pl.program_id(1)
    dinv = dinv_ref[...]
    y = jnp.concatenate([ea_ref[0], eb_ref[0]], axis=1) * dinv
    rid = blk * BM + lax.broadcasted_iota(jnp.int32, (BM, HID), 0)
    m = rid < N

    @pl.when((ph == 0) & (blk == 0))
    def _():
        stats[...] = jnp.zeros_like(stats)
        pooled[...] = jnp.zeros_like(pooled)

    @pl.when(ph == 0)
    def _():
        ym = jnp.where(m, y, 0.0)
        stats[0:1, :] += jnp.sum(ym, axis=0, keepdims=True)
        stats[1:2, :] += jnp.sum(ym * ym, axis=0, keepdims=True)

    @pl.when(ph == 1)
    def _():
        inv_n = jnp.float32(1.0 / N)
        mean = stats[0:1, :] * inv_n
        var = stats[1:2, :] * inv_n - mean * mean
        z = jnp.maximum((y - mean) * lax.rsqrt(var + EPS) * g_ref[...]
                        + b_ref[...], 0.0)
        zm = jnp.where(m, z, 0.0)
        oh = (batch_ref[...] ==
              lax.broadcasted_iota(jnp.int32, (BM, G), 1)).astype(jnp.float32)
        pooled[...] += lax.dot_general(
            oh, zm, (((0,), (0,)), ((), ())),
            preferred_element_type=jnp.float32)

    @pl.when((ph == 1) & (blk == NB - 1))
    def _():
        t = jnp.dot(pooled[...], wn_ref[...],
                    preferred_element_type=jnp.float32) + bn_ref[...]
        lg = jnp.dot(t, wf_ref[...],
                     preferred_element_type=jnp.float32) + bf_ref[...]
        mx = jnp.max(lg, axis=1, keepdims=True)
        lse = jnp.log(jnp.sum(jnp.exp(lg - mx), axis=1, keepdims=True))
        out_ref[...] = lg - mx - lse


def _final_call(agg2, dinv, g2, b2r, batchp, Wn, bnr, Wf, bfr):
    return pl.pallas_call(
        _final_body,
        grid=(2, NB),
        in_specs=[
            pl.BlockSpec((1, BM, HID // 2), lambda p, i: (0, i, 0)),
            pl.BlockSpec((1, BM, HID // 2), lambda p, i: (1, i, 0)),
            pl.BlockSpec((BM, 1), lambda p, i: (i, 0)),
            pl.BlockSpec((1, HID), lambda p, i: (0, 0)),
            pl.BlockSpec((1, HID), lambda p, i: (0, 0)),
            pl.BlockSpec((BM, 1), lambda p, i: (i, 0)),
            pl.BlockSpec((HID, HID), lambda p, i: (0, 0)),
            pl.BlockSpec((1, HID), lambda p, i: (0, 0)),
            pl.BlockSpec((HID, NCLS), lambda p, i: (0, 0)),
            pl.BlockSpec((1, NCLS), lambda p, i: (0, 0)),
        ],
        out_specs=pl.BlockSpec((G, NCLS), lambda p, i: (0, 0)),
        out_shape=jax.ShapeDtypeStruct((G, NCLS), jnp.float32),
        scratch_shapes=[
            pltpu.VMEM((2, HID), jnp.float32),
            pltpu.VMEM((G, HID), jnp.float32),
        ],
    )(agg2, agg2, dinv, g2, b2r, batchp, Wn, bnr, Wf, bfr)


def kernel(x, edge_index, batch, W1, b1, gamma1, beta1, W2, b2, gamma2,
           beta2, Wn, bn, Wf, bf):
    loops = jnp.arange(N, dtype=jnp.int32)
    src = jnp.concatenate([edge_index[0], loops])
    dst = jnp.concatenate([edge_index[1], loops])
    srcr = jnp.pad(src, (0, E_PAD - E_TOT),
                   constant_values=N).reshape(E_PAD // CH, CH)
    dstr = jnp.pad(dst, (0, E_PAD - E_TOT),
                   constant_values=N).reshape(E_PAD // CH, CH)
    dstr_d = jnp.pad(dst, (0, E_PAD_D - E_TOT),
                     constant_values=N).reshape(E_PAD_D // CH, CH)
    xp = jnp.pad(x, ((0, NP - N), (0, 0)))
    batchp = jnp.pad(batch, (0, NP - N), constant_values=G).reshape(NP, 1)

    deg = _deg_call(dstr_d)
    h10, h11, h12, h13, dinv = _lin1_call(xp, W1, deg)
    agg1 = _agg_call(2, [h10, h11, h12, h13], srcr, dstr)
    h2a, h2b = _mid_call(agg1, dinv, gamma1.reshape(2, HID),
                         beta1.reshape(2, HID), W2)
    agg2 = _agg_call(1, [h2a, h2b], srcr, dstr)
    return _final_call(agg2, dinv, gamma2.reshape(1, HID),
                       beta2.reshape(1, HID), batchp, Wn,
                       bn.reshape(1, HID), Wf, bf.reshape(1, NCLS))


# speedup vs baseline: 19.8497x; 3.4065x over previous
import functools

import jax
import jax.numpy as jnp
from jax import lax
from jax.experimental import pallas as pl
from jax.experimental.pallas import tpu as pltpu
from jax.experimental.pallas import tpu_sc as plsc

N = 10000
NP = 10240
G = 128
NCLS = 40
DIN = 128
HID = 128
H2 = 256
EPS = 1e-5

CH = 128
TILES = 16
STRIPE = NP // TILES
E_TOT = 320000 + N
T_CH = 168
E_PAD = TILES * T_CH * CH
DEG_CH = 88
E_PAD_D = 2 * TILES * DEG_CH * CH

BM = 512
NB = NP // BM

_mesh = plsc.VectorSubcoreMesh(core_axis_name="c", subcore_axis_name="s")


def _fill2d(ref, rows, cols, value):
    vec = jnp.full((16,), value, jnp.float32)

    def body(r, _):
        for c in range(cols // 16):
            ref[r, pl.ds(c * 16, 16)] = vec
        return 0

    lax.fori_loop(0, rows, body, 0)


def _deg_body(dstr, deg_out, idx_v, ones_v, deg_sp):
    cid = lax.axis_index("c")
    wid = lax.axis_index("s")
    row0 = wid * STRIPE
    _fill2d(ones_v, CH, 16, 0.0)
    for k in range(STRIPE // CH):
        pltpu.sync_copy(ones_v, deg_sp.at[pl.ds(row0 + k * CH, CH)])
    _fill2d(ones_v, CH, 16, 1.0)
    base = (cid * TILES + wid) * DEG_CH
    pltpu.sync_copy(dstr.at[pl.ds(base, DEG_CH)], idx_v)
    plsc.subcore_barrier()

    def step(j, _):
        pltpu.sync_copy(ones_v, deg_sp.at[idx_v.at[j]], add=True)
        return 0

    lax.fori_loop(0, DEG_CH, step, 0)
    plsc.subcore_barrier()
    pltpu.sync_copy(deg_sp.at[pl.ds(row0, STRIPE)],
                    deg_out.at[cid, pl.ds(row0, STRIPE)])


def _deg_call(dstr):
    return pl.kernel(
        _deg_body,
        out_type=jax.ShapeDtypeStruct((2, NP, 16), jnp.float32),
        mesh=_mesh,
        scratch_types=[
            pltpu.VMEM((DEG_CH, CH), jnp.int32),
            pltpu.VMEM((CH, 16), jnp.float32),
            pltpu.VMEM_SHARED((NP, 16), jnp.float32),
        ],
        compiler_params=pltpu.CompilerParams(use_tc_tiling_on_sc=False),
    )(dstr)


DH = 64
NBUF = 4
NGRP = T_CH // NBUF


def _agg_body(npass, *args):
    hs = args[:2 * npass]
    (srcr, dstr, out, sidx, didx, rows, acc_sp) = args[2 * npass:2 * npass + 7]
    sems = args[2 * npass + 7:]
    gsem = sems[:NBUF]
    ssem = sems[NBUF:]
    cid = lax.axis_index("c")
    wid = lax.axis_index("s")
    row0 = wid * STRIPE
    pltpu.sync_copy(srcr.at[pl.ds(wid * T_CH, T_CH)], sidx)
    pltpu.sync_copy(dstr.at[pl.ds(wid * T_CH, T_CH)], didx)

    def run(h):
        for b in range(NBUF):
            pltpu.async_copy(h.at[sidx.at[b]], rows.at[b], gsem[b])

        def step(i, _):
            for b in range(NBUF):
                pltpu.make_async_copy(
                    h.at[sidx.at[NBUF * i + b]], rows.at[b], gsem[b]).wait()
                pltpu.async_copy(rows.at[b],
                                 acc_sp.at[didx.at[NBUF * i + b]],
                                 ssem[b], add=True)
            for b in range(NBUF):
                jn = jnp.minimum(NBUF * (i + 1) + b, T_CH - 1)
                pltpu.make_async_copy(
                    rows.at[b], acc_sp.at[didx.at[NBUF * i + b]],
                    ssem[b]).wait()
                pltpu.async_copy(h.at[sidx.at[jn]], rows.at[b], gsem[b])
            return 0

        lax.fori_loop(0, NGRP, step, 0)
        for b in range(NBUF):
            pltpu.make_async_copy(
                h.at[sidx.at[T_CH - 1]], rows.at[b], gsem[b]).wait()

    for p in range(npass):
        _fill2d(rows.at[0], CH, DH, 0.0)
        for k in range(STRIPE // CH):
            pltpu.sync_copy(rows.at[0], acc_sp.at[pl.ds(row0 + k * CH, CH)])
        plsc.subcore_barrier()
        pl.when(cid == 0)(functools.partial(run, hs[p]))
        pl.when(cid == 1)(functools.partial(run, hs[npass + p]))
        plsc.subcore_barrier()
        pltpu.sync_copy(
            acc_sp.at[pl.ds(row0, STRIPE)],
            out.at[cid * npass + p, pl.ds(row0, STRIPE)])
        plsc.subcore_barrier()


def _agg_call(npass, hs, srcr, dstr):
    return pl.kernel(
        functools.partial(_agg_body, npass),
        out_type=jax.ShapeDtypeStruct((2 * npass, NP, DH), jnp.float32),
        mesh=_mesh,
        scratch_types=[
            pltpu.VMEM((T_CH, CH), jnp.int32),
            pltpu.VMEM((T_CH, CH), jnp.int32),
            pltpu.VMEM((NBUF, CH, DH), jnp.float32),
            pltpu.VMEM_SHARED((NP, DH), jnp.float32),
        ] + [pltpu.SemaphoreType.DMA] * (2 * NBUF),
        compiler_params=pltpu.CompilerParams(use_tc_tiling_on_sc=False),
    )(*hs, srcr, dstr)


def _lin1_body(x_ref, w_ref, dega_ref, degb_ref, h0_ref, h1_ref, h2_ref,
               h3_ref, dinv_ref):
    deg = dega_ref[0][:, 0:1] + degb_ref[0][:, 0:1]
    dinv = jnp.where(deg > 0, lax.rsqrt(deg), 0.0)
    h = jnp.dot(x_ref[...], w_ref[...], preferred_element_type=jnp.float32)
    hs = h * dinv
    h0_ref[...] = hs[:, 0 * DH:1 * DH]
    h1_ref[...] = hs[:, 1 * DH:2 * DH]
    h2_ref[...] = hs[:, 2 * DH:3 * DH]
    h3_ref[...] = hs[:, 3 * DH:4 * DH]
    dinv_ref[...] = dinv


def _lin1_call(xp, W1, deg):
    hspec = pl.BlockSpec((BM, DH), lambda i: (i, 0))
    hshape = jax.ShapeDtypeStruct((NP, DH), jnp.float32)
    return pl.pallas_call(
        _lin1_body,
        grid=(NB,),
        in_specs=[
            pl.BlockSpec((BM, DIN), lambda i: (i, 0)),
            pl.BlockSpec((DIN, H2), lambda i: (0, 0)),
            pl.BlockSpec((1, BM, 16), lambda i: (0, i, 0)),
            pl.BlockSpec((1, BM, 16), lambda i: (1, i, 0)),
        ],
        out_specs=[hspec, hspec, hspec, hspec,
                   pl.BlockSpec((BM, 1), lambda i: (i, 0))],
        out_shape=[hshape, hshape, hshape, hshape,
                   jax.ShapeDtypeStruct((NP, 1), jnp.float32)],
    )(xp, W1, deg, deg)


def _mid_body(a0_ref, a1_ref, a2_ref, a3_ref, dinv_ref, g_ref, b_ref,
              w2_ref, oa_ref, ob_ref, stats):
    ph = pl.program_id(0)
    blk = pl.program_id(1)
    dinv = dinv_ref[...]
    ya = jnp.concatenate([a0_ref[0], a1_ref[0]], axis=1) * dinv
    yb = jnp.concatenate([a2_ref[0], a3_ref[0]], axis=1) * dinv

    @pl.when((ph == 0) & (blk == 0))
    def _():
        stats[...] = jnp.zeros_like(stats)

    @pl.when(ph == 0)
    def _():
        rid = blk * BM + lax.broadcasted_iota(jnp.int32, (BM, HID), 0)
        m = rid < N
        yam = jnp.where(m, ya, 0.0)
        ybm = jnp.where(m, yb, 0.0)
        stats[0:1, :] += jnp.sum(yam, axis=0, keepdims=True)
        stats[1:2, :] += jnp.sum(ybm, axis=0, keepdims=True)
        stats[2:3, :] += jnp.sum(yam * yam, axis=0, keepdims=True)
        stats[3:4, :] += jnp.sum(ybm * ybm, axis=0, keepdims=True)

    @pl.when(ph == 1)
    def _():
        inv_n = jnp.float32(1.0 / N)
        ma = stats[0:1, :] * inv_n
        mb = stats[1:2, :] * inv_n
        va = stats[2:3, :] * inv_n - ma * ma
        vb = stats[3:4, :] * inv_n - mb * mb
        sa = lax.rsqrt(va + EPS) * g_ref[0:1, :]
        sb = lax.rsqrt(vb + EPS) * g_ref[1:2, :]
        za = jnp.maximum((ya - ma) * sa + b_ref[0:1, :], 0.0)
        zb = jnp.maximum((yb - mb) * sb + b_ref[1:2, :], 0.0)
        t = (jnp.dot(za, w2_ref[:HID, :], preferred_element_type=jnp.float32)
             + jnp.dot(zb, w2_ref[HID:, :],
                       preferred_element_type=jnp.float32))
        ts = t * dinv
        oa_ref[...] = ts[:, :DH]
        ob_ref[...] = ts[:, DH:]


def _mid_call(agg1, dinv, g1, b1r, W2):
    return pl.pallas_call(
        _mid_body,
        grid=(2, NB),
        in_specs=[
            pl.BlockSpec((1, BM, DH), lambda p, i: (0, i, 0)),
            pl.BlockSpec((1, BM, DH), lambda p, i: (1, i, 0)),
            pl.BlockSpec((1, BM, DH), lambda p, i: (2, i, 0)),
            pl.BlockSpec((1, BM, DH), lambda p, i: (3, i, 0)),
            pl.BlockSpec((BM, 1), lambda p, i: (i, 0)),
            pl.BlockSpec((2, HID), lambda p, i: (0, 0)),
            pl.BlockSpec((2, HID), lambda p, i: (0, 0)),
            pl.BlockSpec((H2, HID), lambda p, i: (0, 0)),
        ],
        out_specs=[
            pl.BlockSpec((BM, DH), lambda p, i: (i, 0)),
            pl.BlockSpec((BM, DH), lambda p, i: (i, 0)),
        ],
        out_shape=[
            jax.ShapeDtypeStruct((NP, DH), jnp.float32),
            jax.ShapeDtypeStruct((NP, DH), jnp.float32),
        ],
        scratch_shapes=[pltpu.VMEM((4, HID), jnp.float32)],
    )(agg1, agg1, agg1, agg1, dinv, g1, b1r, W2)


def _final_body(ea_ref, eb_ref, dinv_ref, g_ref, b_ref, batch_ref,
                wn_ref, bn_ref, wf_ref, bf_ref, out_ref, stats, pooled):
    ph = pl.program_id(0)
    blk = pl.program_id(1)
    dinv = dinv_ref[...]
    y = jnp.concatenate([ea_ref[0], eb_ref[0]], axis=1) * dinv
    rid = blk * BM + lax.broadcasted_iota(jnp.int32, (BM, HID), 0)
    m = rid < N

    @pl.when((ph == 0) & (blk == 0))
    def _():
        stats[...] = jnp.zeros_like(stats)
        pooled[...] = jnp.zeros_like(pooled)

    @pl.when(ph == 0)
    def _():
        ym = jnp.where(m, y, 0.0)
        stats[0:1, :] += jnp.sum(ym, axis=0, keepdims=True)
        stats[1:2, :] += jnp.sum(ym * ym, axis=0, keepdims=True)

    @pl.when(ph == 1)
    def _():
        inv_n = jnp.float32(1.0 / N)
        mean = stats[0:1, :] * inv_n
        var = stats[1:2, :] * inv_n - mean * mean
        z = jnp.maximum((y - mean) * lax.rsqrt(var + EPS) * g_ref[...]
                        + b_ref[...], 0.0)
        zm = jnp.where(m, z, 0.0)
        oh = (batch_ref[...] ==
              lax.broadcasted_iota(jnp.int32, (BM, G), 1)).astype(jnp.float32)
        pooled[...] += lax.dot_general(
            oh, zm, (((0,), (0,)), ((), ())),
            preferred_element_type=jnp.float32)

    @pl.when((ph == 1) & (blk == NB - 1))
    def _():
        t = jnp.dot(pooled[...], wn_ref[...],
                    preferred_element_type=jnp.float32) + bn_ref[...]
        lg = jnp.dot(t, wf_ref[...],
                     preferred_element_type=jnp.float32) + bf_ref[...]
        mx = jnp.max(lg, axis=1, keepdims=True)
        lse = jnp.log(jnp.sum(jnp.exp(lg - mx), axis=1, keepdims=True))
        out_ref[...] = lg - mx - lse


def _final_call(agg2, dinv, g2, b2r, batchp, Wn, bnr, Wf, bfr):
    return pl.pallas_call(
        _final_body,
        grid=(2, NB),
        in_specs=[
            pl.BlockSpec((1, BM, DH), lambda p, i: (0, i, 0)),
            pl.BlockSpec((1, BM, DH), lambda p, i: (1, i, 0)),
            pl.BlockSpec((BM, 1), lambda p, i: (i, 0)),
            pl.BlockSpec((1, HID), lambda p, i: (0, 0)),
            pl.BlockSpec((1, HID), lambda p, i: (0, 0)),
            pl.BlockSpec((BM, 1), lambda p, i: (i, 0)),
            pl.BlockSpec((HID, HID), lambda p, i: (0, 0)),
            pl.BlockSpec((1, HID), lambda p, i: (0, 0)),
            pl.BlockSpec((HID, NCLS), lambda p, i: (0, 0)),
            pl.BlockSpec((1, NCLS), lambda p, i: (0, 0)),
        ],
        out_specs=pl.BlockSpec((G, NCLS), lambda p, i: (0, 0)),
        out_shape=jax.ShapeDtypeStruct((G, NCLS), jnp.float32),
        scratch_shapes=[
            pltpu.VMEM((2, HID), jnp.float32),
            pltpu.VMEM((G, HID), jnp.float32),
        ],
    )(agg2, agg2, dinv, g2, b2r, batchp, Wn, bnr, Wf, bfr)


def kernel(x, edge_index, batch, W1, b1, gamma1, beta1, W2, b2, gamma2,
           beta2, Wn, bn, Wf, bf):
    loops = jnp.arange(N, dtype=jnp.int32)
    src = jnp.concatenate([edge_index[0], loops])
    dst = jnp.concatenate([edge_index[1], loops])
    padv = N + (jnp.arange(E_PAD - E_TOT, dtype=jnp.int32) % (NP - N))
    srcr = jnp.concatenate([src, padv]).reshape(E_PAD // CH, CH)
    dstr = jnp.concatenate([dst, padv]).reshape(E_PAD // CH, CH)
    padd = N + (jnp.arange(E_PAD_D - E_TOT, dtype=jnp.int32) % (NP - N))
    dstr_d = jnp.concatenate([dst, padd]).reshape(E_PAD_D // CH, CH)
    xp = jnp.pad(x, ((0, NP - N), (0, 0)))
    batchp = jnp.pad(batch, (0, NP - N), constant_values=G).reshape(NP, 1)

    deg = _deg_call(dstr_d)
    h10, h11, h12, h13, dinv = _lin1_call(xp, W1, deg)
    agg1 = _agg_call(2, [h10, h11, h12, h13], srcr, dstr)
    h2a, h2b = _mid_call(agg1, dinv, gamma1.reshape(2, HID),
                         beta1.reshape(2, HID), W2)
    agg2 = _agg_call(1, [h2a, h2b], srcr, dstr)
    return _final_call(agg2, dinv, gamma2.reshape(1, HID),
                       beta2.reshape(1, HID), batchp, Wn,
                       bn.reshape(1, HID), Wf, bf.reshape(1, NCLS))


# speedup vs baseline: 20.6710x; 1.0414x over previous
import functools

import jax
import jax.numpy as jnp
from jax import lax
from jax.experimental import pallas as pl
from jax.experimental.pallas import tpu as pltpu
from jax.experimental.pallas import tpu_sc as plsc

N = 10000
NP = 10240
G = 128
NCLS = 40
DIN = 128
HID = 128
H2 = 256
EPS = 1e-5

CH = 128
TILES = 16
STRIPE = NP // TILES
E = 320000
T_CH = 160
E_PAD = TILES * T_CH * CH
DEG_CH = T_CH // 2

BM = 512
NB = NP // BM

_mesh = plsc.VectorSubcoreMesh(core_axis_name="c", subcore_axis_name="s")


def _fill2d(ref, rows, cols, value):
    vec = jnp.full((16,), value, jnp.float32)

    def body(r, _):
        for c in range(cols // 16):
            ref[r, pl.ds(c * 16, 16)] = vec
        return 0

    lax.fori_loop(0, rows, body, 0)


def _deg_body(dstr, deg_out, idx_v, ones_v, deg_sp):
    cid = lax.axis_index("c")
    wid = lax.axis_index("s")
    row0 = wid * STRIPE
    _fill2d(ones_v, CH, 16, 0.0)
    for k in range(STRIPE // CH):
        pltpu.sync_copy(ones_v, deg_sp.at[pl.ds(row0 + k * CH, CH)])
    _fill2d(ones_v, CH, 16, 1.0)
    base = (cid * TILES + wid) * DEG_CH
    pltpu.sync_copy(dstr.at[pl.ds(base, DEG_CH)], idx_v)
    plsc.subcore_barrier()

    def step(j, _):
        pltpu.sync_copy(ones_v, deg_sp.at[idx_v.at[j]], add=True)
        return 0

    lax.fori_loop(0, DEG_CH, step, 0)
    plsc.subcore_barrier()
    pltpu.sync_copy(deg_sp.at[pl.ds(row0, STRIPE)],
                    deg_out.at[cid, pl.ds(row0, STRIPE)])


def _deg_call(dstr):
    return pl.kernel(
        _deg_body,
        out_type=jax.ShapeDtypeStruct((2, NP, 16), jnp.float32),
        mesh=_mesh,
        scratch_types=[
            pltpu.VMEM((DEG_CH, CH), jnp.int32),
            pltpu.VMEM((CH, 16), jnp.float32),
            pltpu.VMEM_SHARED((NP, 16), jnp.float32),
        ],
        compiler_params=pltpu.CompilerParams(use_tc_tiling_on_sc=False),
    )(dstr)


DH = 64
NBUF = 4
NGRP = T_CH // NBUF


def _agg_body(npass, *args):
    hs = args[:2 * npass]
    (srcr, dstr, out, sidx, didx, rows, acc_sp) = args[2 * npass:2 * npass + 7]
    sems = args[2 * npass + 7:]
    gsem = sems[:NBUF]
    ssem = sems[NBUF:]
    cid = lax.axis_index("c")
    wid = lax.axis_index("s")
    row0 = wid * STRIPE
    pltpu.sync_copy(srcr.at[pl.ds(wid * T_CH, T_CH)], sidx)
    pltpu.sync_copy(dstr.at[pl.ds(wid * T_CH, T_CH)], didx)

    def run(h):
        for b in range(NBUF):
            pltpu.async_copy(h.at[sidx.at[b]], rows.at[b], gsem[b])

        def step(i, _):
            for b in range(NBUF):
                pltpu.make_async_copy(
                    h.at[sidx.at[NBUF * i + b]], rows.at[b], gsem[b]).wait()
                pltpu.async_copy(rows.at[b],
                                 acc_sp.at[didx.at[NBUF * i + b]],
                                 ssem[b], add=True)
            for b in range(NBUF):
                jn = jnp.minimum(NBUF * (i + 1) + b, T_CH - 1)
                pltpu.make_async_copy(
                    rows.at[b], acc_sp.at[didx.at[NBUF * i + b]],
                    ssem[b]).wait()
                pltpu.async_copy(h.at[sidx.at[jn]], rows.at[b], gsem[b])
            return 0

        lax.fori_loop(0, NGRP, step, 0)
        for b in range(NBUF):
            pltpu.make_async_copy(
                h.at[sidx.at[T_CH - 1]], rows.at[b], gsem[b]).wait()

    def init_acc(h):
        pltpu.sync_copy(h.at[pl.ds(row0, STRIPE)],
                        acc_sp.at[pl.ds(row0, STRIPE)])

    for p in range(npass):
        pl.when(cid == 0)(functools.partial(init_acc, hs[p]))
        pl.when(cid == 1)(functools.partial(init_acc, hs[npass + p]))
        plsc.subcore_barrier()
        pl.when(cid == 0)(functools.partial(run, hs[p]))
        pl.when(cid == 1)(functools.partial(run, hs[npass + p]))
        plsc.subcore_barrier()
        pltpu.sync_copy(
            acc_sp.at[pl.ds(row0, STRIPE)],
            out.at[cid * npass + p, pl.ds(row0, STRIPE)])
        plsc.subcore_barrier()


def _agg_call(npass, hs, srcr, dstr):
    return pl.kernel(
        functools.partial(_agg_body, npass),
        out_type=jax.ShapeDtypeStruct((2 * npass, NP, DH), jnp.float32),
        mesh=_mesh,
        scratch_types=[
            pltpu.VMEM((T_CH, CH), jnp.int32),
            pltpu.VMEM((T_CH, CH), jnp.int32),
            pltpu.VMEM((NBUF, CH, DH), jnp.float32),
            pltpu.VMEM_SHARED((NP, DH), jnp.float32),
        ] + [pltpu.SemaphoreType.DMA] * (2 * NBUF),
        compiler_params=pltpu.CompilerParams(use_tc_tiling_on_sc=False),
    )(*hs, srcr, dstr)


def _lin1_body(x_ref, w_ref, dega_ref, degb_ref, h0_ref, h1_ref, h2_ref,
               h3_ref, dinv_ref):
    deg = dega_ref[0][:, 0:1] + degb_ref[0][:, 0:1] + 1.0
    dinv = lax.rsqrt(deg)
    h = jnp.dot(x_ref[...], w_ref[...], preferred_element_type=jnp.float32)
    hs = h * dinv
    h0_ref[...] = hs[:, 0 * DH:1 * DH]
    h1_ref[...] = hs[:, 1 * DH:2 * DH]
    h2_ref[...] = hs[:, 2 * DH:3 * DH]
    h3_ref[...] = hs[:, 3 * DH:4 * DH]
    dinv_ref[...] = dinv


def _lin1_call(xp, W1, deg):
    hspec = pl.BlockSpec((BM, DH), lambda i: (i, 0))
    hshape = jax.ShapeDtypeStruct((NP, DH), jnp.float32)
    return pl.pallas_call(
        _lin1_body,
        grid=(NB,),
        in_specs=[
            pl.BlockSpec((BM, DIN), lambda i: (i, 0)),
            pl.BlockSpec((DIN, H2), lambda i: (0, 0)),
            pl.BlockSpec((1, BM, 16), lambda i: (0, i, 0)),
            pl.BlockSpec((1, BM, 16), lambda i: (1, i, 0)),
        ],
        out_specs=[hspec, hspec, hspec, hspec,
                   pl.BlockSpec((BM, 1), lambda i: (i, 0))],
        out_shape=[hshape, hshape, hshape, hshape,
                   jax.ShapeDtypeStruct((NP, 1), jnp.float32)],
    )(xp, W1, deg, deg)


def _mid_body(a0_ref, a1_ref, a2_ref, a3_ref, dinv_ref, g_ref, b_ref,
              w2_ref, oa_ref, ob_ref, stats):
    ph = pl.program_id(0)
    blk = pl.program_id(1)
    dinv = dinv_ref[...]
    ya = jnp.concatenate([a0_ref[0], a1_ref[0]], axis=1) * dinv
    yb = jnp.concatenate([a2_ref[0], a3_ref[0]], axis=1) * dinv

    @pl.when((ph == 0) & (blk == 0))
    def _():
        stats[...] = jnp.zeros_like(stats)

    @pl.when(ph == 0)
    def _():
        rid = blk * BM + lax.broadcasted_iota(jnp.int32, (BM, HID), 0)
        m = rid < N
        yam = jnp.where(m, ya, 0.0)
        ybm = jnp.where(m, yb, 0.0)
        stats[0:1, :] += jnp.sum(yam, axis=0, keepdims=True)
        stats[1:2, :] += jnp.sum(ybm, axis=0, keepdims=True)
        stats[2:3, :] += jnp.sum(yam * yam, axis=0, keepdims=True)
        stats[3:4, :] += jnp.sum(ybm * ybm, axis=0, keepdims=True)

    @pl.when(ph == 1)
    def _():
        inv_n = jnp.float32(1.0 / N)
        ma = stats[0:1, :] * inv_n
        mb = stats[1:2, :] * inv_n
        va = stats[2:3, :] * inv_n - ma * ma
        vb = stats[3:4, :] * inv_n - mb * mb
        sa = lax.rsqrt(va + EPS) * g_ref[0:1, :]
        sb = lax.rsqrt(vb + EPS) * g_ref[1:2, :]
        za = jnp.maximum((ya - ma) * sa + b_ref[0:1, :], 0.0)
        zb = jnp.maximum((yb - mb) * sb + b_ref[1:2, :], 0.0)
        t = (jnp.dot(za, w2_ref[:HID, :], preferred_element_type=jnp.float32)
             + jnp.dot(zb, w2_ref[HID:, :],
                       preferred_element_type=jnp.float32))
        ts = t * dinv
        oa_ref[...] = ts[:, :DH]
        ob_ref[...] = ts[:, DH:]


def _mid_call(agg1, dinv, g1, b1r, W2):
    return pl.pallas_call(
        _mid_body,
        grid=(2, NB),
        in_specs=[
            pl.BlockSpec((1, BM, DH), lambda p, i: (0, i, 0)),
            pl.BlockSpec((1, BM, DH), lambda p, i: (1, i, 0)),
            pl.BlockSpec((1, BM, DH), lambda p, i: (2, i, 0)),
            pl.BlockSpec((1, BM, DH), lambda p, i: (3, i, 0)),
            pl.BlockSpec((BM, 1), lambda p, i: (i, 0)),
            pl.BlockSpec((2, HID), lambda p, i: (0, 0)),
            pl.BlockSpec((2, HID), lambda p, i: (0, 0)),
            pl.BlockSpec((H2, HID), lambda p, i: (0, 0)),
        ],
        out_specs=[
            pl.BlockSpec((BM, DH), lambda p, i: (i, 0)),
            pl.BlockSpec((BM, DH), lambda p, i: (i, 0)),
        ],
        out_shape=[
            jax.ShapeDtypeStruct((NP, DH), jnp.float32),
            jax.ShapeDtypeStruct((NP, DH), jnp.float32),
        ],
        scratch_shapes=[pltpu.VMEM((4, HID), jnp.float32)],
    )(agg1, agg1, agg1, agg1, dinv, g1, b1r, W2)


def _final_body(ea_ref, eb_ref, dinv_ref, g_ref, b_ref, batch_ref,
                wn_ref, bn_ref, wf_ref, bf_ref, out_ref, stats, pooled):
    ph = pl.program_id(0)
    blk = pl.program_id(1)
    dinv = dinv_ref[...]
    y = jnp.concatenate([ea_ref[0], eb_ref[0]], axis=1) * dinv
    rid = blk * BM + lax.broadcasted_iota(jnp.int32, (BM, HID), 0)
    m = rid < N

    @pl.when((ph == 0) & (blk == 0))
    def _():
        stats[...] = jnp.zeros_like(stats)
        pooled[...] = jnp.zeros_like(pooled)

    @pl.when(ph == 0)
    def _():
        ym = jnp.where(m, y, 0.0)
        stats[0:1, :] += jnp.sum(ym, axis=0, keepdims=True)
        stats[1:2, :] += jnp.sum(ym * ym, axis=0, keepdims=True)

    @pl.when(ph == 1)
    def _():
        inv_n = jnp.float32(1.0 / N)
        mean = stats[0:1, :] * inv_n
        var = stats[1:2, :] * inv_n - mean * mean
        z = jnp.maximum((y - mean) * lax.rsqrt(var + EPS) * g_ref[...]
                        + b_ref[...], 0.0)
        zm = jnp.where(m, z, 0.0)
        oh = (batch_ref[...] ==
              lax.broadcasted_iota(jnp.int32, (BM, G), 1)).astype(jnp.float32)
        pooled[...] += lax.dot_general(
            oh, zm, (((0,), (0,)), ((), ())),
            preferred_element_type=jnp.float32)

    @pl.when((ph == 1) & (blk == NB - 1))
    def _():
        t = jnp.dot(pooled[...], wn_ref[...],
                    preferred_element_type=jnp.float32) + bn_ref[...]
        lg = jnp.dot(t, wf_ref[...],
                     preferred_element_type=jnp.float32) + bf_ref[...]
        mx = jnp.max(lg, axis=1, keepdims=True)
        lse = jnp.log(jnp.sum(jnp.exp(lg - mx), axis=1, keepdims=True))
        out_ref[...] = lg - mx - lse


def _final_call(agg2, dinv, g2, b2r, batchp, Wn, bnr, Wf, bfr):
    return pl.pallas_call(
        _final_body,
        grid=(2, NB),
        in_specs=[
            pl.BlockSpec((1, BM, DH), lambda p, i: (0, i, 0)),
            pl.BlockSpec((1, BM, DH), lambda p, i: (1, i, 0)),
            pl.BlockSpec((BM, 1), lambda p, i: (i, 0)),
            pl.BlockSpec((1, HID), lambda p, i: (0, 0)),
            pl.BlockSpec((1, HID), lambda p, i: (0, 0)),
            pl.BlockSpec((BM, 1), lambda p, i: (i, 0)),
            pl.BlockSpec((HID, HID), lambda p, i: (0, 0)),
            pl.BlockSpec((1, HID), lambda p, i: (0, 0)),
            pl.BlockSpec((HID, NCLS), lambda p, i: (0, 0)),
            pl.BlockSpec((1, NCLS), lambda p, i: (0, 0)),
        ],
        out_specs=pl.BlockSpec((G, NCLS), lambda p, i: (0, 0)),
        out_shape=jax.ShapeDtypeStruct((G, NCLS), jnp.float32),
        scratch_shapes=[
            pltpu.VMEM((2, HID), jnp.float32),
            pltpu.VMEM((G, HID), jnp.float32),
        ],
    )(agg2, agg2, dinv, g2, b2r, batchp, Wn, bnr, Wf, bfr)


def kernel(x, edge_index, batch, W1, b1, gamma1, beta1, W2, b2, gamma2,
           beta2, Wn, bn, Wf, bf):
    padv = N + (jnp.arange(E_PAD - E, dtype=jnp.int32) % (NP - N))
    srcr = jnp.concatenate([edge_index[0], padv]).reshape(E_PAD // CH, CH)
    dstr = jnp.concatenate([edge_index[1], padv]).reshape(E_PAD // CH, CH)
    xp = jnp.pad(x, ((0, NP - N), (0, 0)))
    batchp = jnp.pad(batch, (0, NP - N), constant_values=G).reshape(NP, 1)

    deg = _deg_call(dstr)
    h10, h11, h12, h13, dinv = _lin1_call(xp, W1, deg)
    agg1 = _agg_call(2, [h10, h11, h12, h13], srcr, dstr)
    h2a, h2b = _mid_call(agg1, dinv, gamma1.reshape(2, HID),
                         beta1.reshape(2, HID), W2)
    agg2 = _agg_call(1, [h2a, h2b], srcr, dstr)
    return _final_call(agg2, dinv, gamma2.reshape(1, HID),
                       beta2.reshape(1, HID), batchp, Wn,
                       bn.reshape(1, HID), Wf, bf.reshape(1, NCLS))


# speedup vs baseline: 22.2767x; 1.0777x over previous
import functools

import jax
import jax.numpy as jnp
from jax import lax
from jax.experimental import pallas as pl
from jax.experimental.pallas import tpu as pltpu
from jax.experimental.pallas import tpu_sc as plsc

N = 10000
NP = 10240
G = 128
NCLS = 40
DIN = 128
HID = 128
H2 = 256
EPS = 1e-5

CH = 128
TILES = 16
STRIPE = NP // TILES
E = 320000
T_CH = 160
E_PAD = TILES * T_CH * CH
DEG_CH = T_CH // 2

BM = 512
NB = NP // BM

_mesh = plsc.VectorSubcoreMesh(core_axis_name="c", subcore_axis_name="s")


def _fill2d(ref, rows, cols, value):
    vec = jnp.full((16,), value, jnp.float32)

    def body(r, _):
        for c in range(cols // 16):
            ref[r, pl.ds(c * 16, 16)] = vec
        return 0

    lax.fori_loop(0, rows, body, 0)


def _deg_body(dstr, deg_out, idx_v, ones_v, deg_sp):
    cid = lax.axis_index("c")
    wid = lax.axis_index("s")
    row0 = wid * STRIPE
    _fill2d(ones_v, CH, 16, 0.0)
    for k in range(STRIPE // CH):
        pltpu.sync_copy(ones_v, deg_sp.at[pl.ds(row0 + k * CH, CH)])
    _fill2d(ones_v, CH, 16, 1.0)
    base = (cid * TILES + wid) * DEG_CH
    pltpu.sync_copy(dstr.at[pl.ds(base, DEG_CH)], idx_v)
    plsc.subcore_barrier()

    def step(j, _):
        pltpu.sync_copy(ones_v, deg_sp.at[idx_v.at[j]], add=True)
        return 0

    lax.fori_loop(0, DEG_CH, step, 0)
    plsc.subcore_barrier()
    pltpu.sync_copy(deg_sp.at[pl.ds(row0, STRIPE)],
                    deg_out.at[cid, pl.ds(row0, STRIPE)])


def _deg_call(dstr):
    return pl.kernel(
        _deg_body,
        out_type=jax.ShapeDtypeStruct((2, NP, 16), jnp.float32),
        mesh=_mesh,
        scratch_types=[
            pltpu.VMEM((DEG_CH, CH), jnp.int32),
            pltpu.VMEM((CH, 16), jnp.float32),
            pltpu.VMEM_SHARED((NP, 16), jnp.float32),
        ],
        compiler_params=pltpu.CompilerParams(use_tc_tiling_on_sc=False),
    )(dstr)


DH = 64
NBUF = 4
NGRP = T_CH // NBUF


def _agg_body(npass, *args):
    hs = args[:2 * npass]
    (srcr, dstr, out, sidx, didx, rows, acc_sp) = args[2 * npass:2 * npass + 7]
    sems = args[2 * npass + 7:]
    gsem = sems[:NBUF]
    ssem = sems[NBUF:]
    cid = lax.axis_index("c")
    wid = lax.axis_index("s")
    row0 = wid * STRIPE
    pltpu.sync_copy(srcr.at[pl.ds(wid * T_CH, T_CH)], sidx)
    pltpu.sync_copy(dstr.at[pl.ds(wid * T_CH, T_CH)], didx)

    def run(h):
        for b in range(NBUF):
            pltpu.async_copy(h.at[sidx.at[b]], rows.at[b], gsem[b])

        def step(i, _):
            for b in range(NBUF):
                pltpu.make_async_copy(
                    h.at[sidx.at[NBUF * i + b]], rows.at[b], gsem[b]).wait()
                pltpu.async_copy(rows.at[b],
                                 acc_sp.at[didx.at[NBUF * i + b]],
                                 ssem[b], add=True)
            for b in range(NBUF):
                jn = jnp.minimum(NBUF * (i + 1) + b, T_CH - 1)
                pltpu.make_async_copy(
                    rows.at[b], acc_sp.at[didx.at[NBUF * i + b]],
                    ssem[b]).wait()
                pltpu.async_copy(h.at[sidx.at[jn]], rows.at[b], gsem[b])
            return 0

        lax.fori_loop(0, NGRP, step, 0)
        for b in range(NBUF):
            pltpu.make_async_copy(
                h.at[sidx.at[T_CH - 1]], rows.at[b], gsem[b]).wait()

    def init_acc(h):
        pltpu.sync_copy(h.at[pl.ds(row0, STRIPE)],
                        acc_sp.at[pl.ds(row0, STRIPE)])

    for p in range(npass):
        pl.when(cid == 0)(functools.partial(init_acc, hs[p]))
        pl.when(cid == 1)(functools.partial(init_acc, hs[npass + p]))
        plsc.subcore_barrier()
        pl.when(cid == 0)(functools.partial(run, hs[p]))
        pl.when(cid == 1)(functools.partial(run, hs[npass + p]))
        plsc.subcore_barrier()
        g = cid * npass + p
        pltpu.sync_copy(
            acc_sp.at[pl.ds(row0, STRIPE)],
            out.at[g // 2, pl.ds(row0, STRIPE), pl.ds((g % 2) * DH, DH)])
        plsc.subcore_barrier()


def _agg_call(npass, hs, srcr, dstr):
    return pl.kernel(
        functools.partial(_agg_body, npass),
        out_type=jax.ShapeDtypeStruct((npass, NP, 2 * DH), jnp.float32),
        mesh=_mesh,
        scratch_types=[
            pltpu.VMEM((T_CH, CH), jnp.int32),
            pltpu.VMEM((T_CH, CH), jnp.int32),
            pltpu.VMEM((NBUF, CH, DH), jnp.float32),
            pltpu.VMEM_SHARED((NP, DH), jnp.float32),
        ] + [pltpu.SemaphoreType.DMA] * (2 * NBUF),
        compiler_params=pltpu.CompilerParams(use_tc_tiling_on_sc=False),
    )(*hs, srcr, dstr)


def _lin1_body(x_ref, w_ref, dega_ref, degb_ref, h0_ref, h1_ref, h2_ref,
               h3_ref, dinv_ref):
    deg = dega_ref[0][:, 0:1] + degb_ref[0][:, 0:1] + 1.0
    dinv = lax.rsqrt(deg)
    h = jnp.dot(x_ref[...], w_ref[...], preferred_element_type=jnp.float32)
    hs = h * dinv
    h0_ref[...] = hs[:, 0 * DH:1 * DH]
    h1_ref[...] = hs[:, 1 * DH:2 * DH]
    h2_ref[...] = hs[:, 2 * DH:3 * DH]
    h3_ref[...] = hs[:, 3 * DH:4 * DH]
    dinv_ref[...] = dinv


def _lin1_call(xp, W1, deg):
    hspec = pl.BlockSpec((BM, DH), lambda i: (i, 0))
    hshape = jax.ShapeDtypeStruct((NP, DH), jnp.float32)
    return pl.pallas_call(
        _lin1_body,
        grid=(NB,),
        in_specs=[
            pl.BlockSpec((BM, DIN), lambda i: (i, 0)),
            pl.BlockSpec((DIN, H2), lambda i: (0, 0)),
            pl.BlockSpec((1, BM, 16), lambda i: (0, i, 0)),
            pl.BlockSpec((1, BM, 16), lambda i: (1, i, 0)),
        ],
        out_specs=[hspec, hspec, hspec, hspec,
                   pl.BlockSpec((BM, 1), lambda i: (i, 0))],
        out_shape=[hshape, hshape, hshape, hshape,
                   jax.ShapeDtypeStruct((NP, 1), jnp.float32)],
    )(xp, W1, deg, deg)


def _mid_body(atop_ref, abot_ref, dinv_ref, g_ref, b_ref,
              w2_ref, oa_ref, ob_ref, stats):
    ph = pl.program_id(0)
    blk = pl.program_id(1)
    dinv = dinv_ref[...]
    ya = atop_ref[0] * dinv
    yb = abot_ref[0] * dinv

    @pl.when((ph == 0) & (blk == 0))
    def _():
        stats[...] = jnp.zeros_like(stats)

    @pl.when(ph == 0)
    def _():
        rid = blk * BM + lax.broadcasted_iota(jnp.int32, (BM, HID), 0)
        m = rid < N
        yam = jnp.where(m, ya, 0.0)
        ybm = jnp.where(m, yb, 0.0)
        stats[0:1, :] += jnp.sum(yam, axis=0, keepdims=True)
        stats[1:2, :] += jnp.sum(ybm, axis=0, keepdims=True)
        stats[2:3, :] += jnp.sum(yam * yam, axis=0, keepdims=True)
        stats[3:4, :] += jnp.sum(ybm * ybm, axis=0, keepdims=True)

    @pl.when(ph == 1)
    def _():
        inv_n = jnp.float32(1.0 / N)
        ma = stats[0:1, :] * inv_n
        mb = stats[1:2, :] * inv_n
        va = stats[2:3, :] * inv_n - ma * ma
        vb = stats[3:4, :] * inv_n - mb * mb
        sa = lax.rsqrt(va + EPS) * g_ref[0:1, :]
        sb = lax.rsqrt(vb + EPS) * g_ref[1:2, :]
        za = jnp.maximum((ya - ma) * sa + b_ref[0:1, :], 0.0)
        zb = jnp.maximum((yb - mb) * sb + b_ref[1:2, :], 0.0)
        t = (jnp.dot(za, w2_ref[:HID, :], preferred_element_type=jnp.float32)
             + jnp.dot(zb, w2_ref[HID:, :],
                       preferred_element_type=jnp.float32))
        ts = t * dinv
        oa_ref[...] = ts[:, :DH]
        ob_ref[...] = ts[:, DH:]


def _mid_call(agg1, dinv, g1, b1r, W2):
    return pl.pallas_call(
        _mid_body,
        grid=(2, NB),
        in_specs=[
            pl.BlockSpec((1, BM, HID), lambda p, i: (0, i, 0)),
            pl.BlockSpec((1, BM, HID), lambda p, i: (1, i, 0)),
            pl.BlockSpec((BM, 1), lambda p, i: (i, 0)),
            pl.BlockSpec((2, HID), lambda p, i: (0, 0)),
            pl.BlockSpec((2, HID), lambda p, i: (0, 0)),
            pl.BlockSpec((H2, HID), lambda p, i: (0, 0)),
        ],
        out_specs=[
            pl.BlockSpec((BM, DH), lambda p, i: (p * i, 0)),
            pl.BlockSpec((BM, DH), lambda p, i: (p * i, 0)),
        ],
        out_shape=[
            jax.ShapeDtypeStruct((NP, DH), jnp.float32),
            jax.ShapeDtypeStruct((NP, DH), jnp.float32),
        ],
        scratch_shapes=[pltpu.VMEM((4, HID), jnp.float32)],
    )(agg1, agg1, dinv, g1, b1r, W2)


def _final_body(e_ref, dinv_ref, g_ref, b_ref, batch_ref,
                wn_ref, bn_ref, wf_ref, bf_ref, out_ref, stats, pooled):
    ph = pl.program_id(0)
    blk = pl.program_id(1)
    dinv = dinv_ref[...]
    y = e_ref[0] * dinv
    rid = blk * BM + lax.broadcasted_iota(jnp.int32, (BM, HID), 0)
    m = rid < N

    @pl.when((ph == 0) & (blk == 0))
    def _():
        stats[...] = jnp.zeros_like(stats)
        pooled[...] = jnp.zeros_like(pooled)

    @pl.when(ph == 0)
    def _():
        ym = jnp.where(m, y, 0.0)
        stats[0:1, :] += jnp.sum(ym, axis=0, keepdims=True)
        stats[1:2, :] += jnp.sum(ym * ym, axis=0, keepdims=True)

    @pl.when(ph == 1)
    def _():
        inv_n = jnp.float32(1.0 / N)
        mean = stats[0:1, :] * inv_n
        var = stats[1:2, :] * inv_n - mean * mean
        z = jnp.maximum((y - mean) * lax.rsqrt(var + EPS) * g_ref[...]
                        + b_ref[...], 0.0)
        zm = jnp.where(m, z, 0.0)
        oh = (batch_ref[...] ==
              lax.broadcasted_iota(jnp.int32, (BM, G), 1)).astype(jnp.float32)
        pooled[...] += lax.dot_general(
            oh, zm, (((0,), (0,)), ((), ())),
            preferred_element_type=jnp.float32)

    @pl.when((ph == 1) & (blk == NB - 1))
    def _():
        t = jnp.dot(pooled[...], wn_ref[...],
                    preferred_element_type=jnp.float32) + bn_ref[...]
        lg = jnp.dot(t, wf_ref[...],
                     preferred_element_type=jnp.float32) + bf_ref[...]
        mx = jnp.max(lg, axis=1, keepdims=True)
        lse = jnp.log(jnp.sum(jnp.exp(lg - mx), axis=1, keepdims=True))
        out_ref[...] = lg - mx - lse


def _final_call(agg2, dinv, g2, b2r, batchp, Wn, bnr, Wf, bfr):
    return pl.pallas_call(
        _final_body,
        grid=(2, NB),
        in_specs=[
            pl.BlockSpec((1, BM, HID), lambda p, i: (0, i, 0)),
            pl.BlockSpec((BM, 1), lambda p, i: (i, 0)),
            pl.BlockSpec((1, HID), lambda p, i: (0, 0)),
            pl.BlockSpec((1, HID), lambda p, i: (0, 0)),
            pl.BlockSpec((BM, 1), lambda p, i: (i, 0)),
            pl.BlockSpec((HID, HID), lambda p, i: (0, 0)),
            pl.BlockSpec((1, HID), lambda p, i: (0, 0)),
            pl.BlockSpec((HID, NCLS), lambda p, i: (0, 0)),
            pl.BlockSpec((1, NCLS), lambda p, i: (0, 0)),
        ],
        out_specs=pl.BlockSpec((G, NCLS), lambda p, i: (0, 0)),
        out_shape=jax.ShapeDtypeStruct((G, NCLS), jnp.float32),
        scratch_shapes=[
            pltpu.VMEM((2, HID), jnp.float32),
            pltpu.VMEM((G, HID), jnp.float32),
        ],
    )(agg2, dinv, g2, b2r, batchp, Wn, bnr, Wf, bfr)


def kernel(x, edge_index, batch, W1, b1, gamma1, beta1, W2, b2, gamma2,
           beta2, Wn, bn, Wf, bf):
    padv = N + (jnp.arange(E_PAD - E, dtype=jnp.int32) % (NP - N))
    srcr = jnp.concatenate([edge_index[0], padv]).reshape(E_PAD // CH, CH)
    dstr = jnp.concatenate([edge_index[1], padv]).reshape(E_PAD // CH, CH)
    xp = jnp.pad(x, ((0, NP - N), (0, 0)))
    batchp = jnp.pad(batch, (0, NP - N), constant_values=G).reshape(NP, 1)

    deg = _deg_call(dstr)
    h10, h11, h12, h13, dinv = _lin1_call(xp, W1, deg)
    agg1 = _agg_call(2, [h10, h11, h12, h13], srcr, dstr)
    h2a, h2b = _mid_call(agg1, dinv, gamma1.reshape(2, HID),
                         beta1.reshape(2, HID), W2)
    agg2 = _agg_call(1, [h2a, h2b], srcr, dstr)
    return _final_call(agg2, dinv, gamma2.reshape(1, HID),
                       beta2.reshape(1, HID), batchp, Wn,
                       bn.reshape(1, HID), Wf, bf.reshape(1, NCLS))


# speedup vs baseline: 23.1113x; 1.0375x over previous
import functools

import jax
import jax.numpy as jnp
from jax import lax
from jax.experimental import pallas as pl
from jax.experimental.pallas import tpu as pltpu
from jax.experimental.pallas import tpu_sc as plsc

N = 10000
NP = 10240
G = 128
NCLS = 40
DIN = 128
HID = 128
H2 = 256
EPS = 1e-5

CH = 128
TILES = 16
STRIPE = NP // TILES
E = 320000
T_CH = 160
E_PAD = TILES * T_CH * CH
DEG_CH = T_CH // 2

BM = 512
NB = NP // BM

_mesh = plsc.VectorSubcoreMesh(core_axis_name="c", subcore_axis_name="s")


def _fill2d(ref, rows, cols, value):
    vec = jnp.full((16,), value, jnp.float32)

    def body(r, _):
        for c in range(cols // 16):
            ref[r, pl.ds(c * 16, 16)] = vec
        return 0

    lax.fori_loop(0, rows, body, 0)


def _deg_body(dstr, deg_out, idx_v, ones_v, deg_sp):
    cid = lax.axis_index("c")
    wid = lax.axis_index("s")
    row0 = wid * STRIPE
    _fill2d(ones_v, CH, 16, 0.0)
    for k in range(STRIPE // CH):
        pltpu.sync_copy(ones_v, deg_sp.at[pl.ds(row0 + k * CH, CH)])
    _fill2d(ones_v, CH, 16, 1.0)
    base = (cid * TILES + wid) * DEG_CH
    pltpu.sync_copy(dstr.at[pl.ds(base, DEG_CH)], idx_v)
    plsc.subcore_barrier()

    def step(j, _):
        pltpu.sync_copy(ones_v, deg_sp.at[idx_v.at[j]], add=True)
        return 0

    lax.fori_loop(0, DEG_CH, step, 0)
    plsc.subcore_barrier()
    pltpu.sync_copy(deg_sp.at[pl.ds(row0, STRIPE)],
                    deg_out.at[cid, pl.ds(row0, STRIPE)])


def _deg_call(dstr):
    return pl.kernel(
        _deg_body,
        out_type=jax.ShapeDtypeStruct((2, NP, 16), jnp.float32),
        mesh=_mesh,
        scratch_types=[
            pltpu.VMEM((DEG_CH, CH), jnp.int32),
            pltpu.VMEM((CH, 16), jnp.float32),
            pltpu.VMEM_SHARED((NP, 16), jnp.float32),
        ],
        compiler_params=pltpu.CompilerParams(use_tc_tiling_on_sc=False),
    )(dstr)


DH = 64
NBUF = 4
NGRP = T_CH // NBUF


def _agg_body(npass, *args):
    hg = args[:2]
    iids = args[2:4]
    srs = args[4:4 + 2 * npass]
    (dstr, out, sidx, didx, rows, acc_sp) = args[4 + 2 * npass:10 + 2 * npass]
    sems = args[10 + 2 * npass:]
    gsem = sems[:NBUF]
    ssem = sems[NBUF:]
    cid = lax.axis_index("c")
    wid = lax.axis_index("s")
    row0 = wid * STRIPE
    pltpu.sync_copy(dstr.at[pl.ds(wid * T_CH, T_CH)], didx)

    def run(h, sr):
        pltpu.sync_copy(sr.at[pl.ds(wid * T_CH, T_CH)], sidx)
        for b in range(NBUF):
            pltpu.async_copy(h.at[sidx.at[b]], rows.at[b], gsem[b])

        def step(i, _):
            for b in range(NBUF):
                pltpu.make_async_copy(
                    h.at[sidx.at[NBUF * i + b]], rows.at[b], gsem[b]).wait()
                pltpu.async_copy(rows.at[b],
                                 acc_sp.at[didx.at[NBUF * i + b]],
                                 ssem[b], add=True)
            for b in range(NBUF):
                jn = jnp.minimum(NBUF * (i + 1) + b, T_CH - 1)
                pltpu.make_async_copy(
                    rows.at[b], acc_sp.at[didx.at[NBUF * i + b]],
                    ssem[b]).wait()
                pltpu.async_copy(h.at[sidx.at[jn]], rows.at[b], gsem[b])
            return 0

        lax.fori_loop(0, NGRP, step, 0)
        for b in range(NBUF):
            pltpu.make_async_copy(
                h.at[sidx.at[T_CH - 1]], rows.at[b], gsem[b]).wait()

    def init_acc(h, iid):
        nk = STRIPE // CH
        pltpu.sync_copy(iid.at[pl.ds(wid * nk, nk)], sidx.at[pl.ds(0, nk)])
        for k in range(nk):
            pltpu.async_copy(h.at[sidx.at[k]], rows.at[0], gsem[0]).wait()
            pltpu.sync_copy(rows.at[0],
                            acc_sp.at[pl.ds(row0 + k * CH, CH)])

    for p in range(npass):
        for c in range(2):
            g = c * npass + p
            pl.when(cid == c)(functools.partial(init_acc, hg[c],
                                                iids[g % 2]))
        plsc.subcore_barrier()
        for c in range(2):
            pl.when(cid == c)(functools.partial(run, hg[c],
                                                srs[c * npass + p]))
        plsc.subcore_barrier()
        g = cid * npass + p
        pltpu.sync_copy(
            acc_sp.at[pl.ds(row0, STRIPE)],
            out.at[g // 2, pl.ds(row0, STRIPE), pl.ds((g % 2) * DH, DH)])
        plsc.subcore_barrier()


def _agg_call(npass, hg, iids, srs, dstr):
    return pl.kernel(
        functools.partial(_agg_body, npass),
        out_type=jax.ShapeDtypeStruct((npass, NP, 2 * DH), jnp.float32),
        mesh=_mesh,
        scratch_types=[
            pltpu.VMEM((T_CH, CH), jnp.int32),
            pltpu.VMEM((T_CH, CH), jnp.int32),
            pltpu.VMEM((NBUF, CH, DH), jnp.float32),
            pltpu.VMEM_SHARED((NP, DH), jnp.float32),
        ] + [pltpu.SemaphoreType.DMA] * (2 * NBUF),
        compiler_params=pltpu.CompilerParams(use_tc_tiling_on_sc=False),
    )(*hg, *iids, *srs, dstr)


def _lin1_body(x_ref, w_ref, dega_ref, degb_ref, ha_ref, hb_ref, dinv_ref):
    deg = dega_ref[0][:, 0:1] + degb_ref[0][:, 0:1] + 1.0
    dinv = lax.rsqrt(deg)
    h = jnp.dot(x_ref[...], w_ref[...], preferred_element_type=jnp.float32)
    hs = h * dinv
    ha_ref[...] = hs[:, :HID]
    hb_ref[...] = hs[:, HID:]
    dinv_ref[...] = dinv


def _lin1_call(xp, W1, deg):
    hspec = pl.BlockSpec((BM, HID), lambda i: (i, 0))
    hshape = jax.ShapeDtypeStruct((NP, HID), jnp.float32)
    return pl.pallas_call(
        _lin1_body,
        grid=(NB,),
        in_specs=[
            pl.BlockSpec((BM, DIN), lambda i: (i, 0)),
            pl.BlockSpec((DIN, H2), lambda i: (0, 0)),
            pl.BlockSpec((1, BM, 16), lambda i: (0, i, 0)),
            pl.BlockSpec((1, BM, 16), lambda i: (1, i, 0)),
        ],
        out_specs=[hspec, hspec,
                   pl.BlockSpec((BM, 1), lambda i: (i, 0))],
        out_shape=[hshape, hshape,
                   jax.ShapeDtypeStruct((NP, 1), jnp.float32)],
    )(xp, W1, deg, deg)


def _mid_body(atop_ref, abot_ref, dinv_ref, g_ref, b_ref,
              w2_ref, o_ref, stats):
    ph = pl.program_id(0)
    blk = pl.program_id(1)
    dinv = dinv_ref[...]
    ya = atop_ref[0] * dinv
    yb = abot_ref[0] * dinv

    @pl.when((ph == 0) & (blk == 0))
    def _():
        stats[...] = jnp.zeros_like(stats)

    @pl.when(ph == 0)
    def _():
        rid = blk * BM + lax.broadcasted_iota(jnp.int32, (BM, HID), 0)
        m = rid < N
        yam = jnp.where(m, ya, 0.0)
        ybm = jnp.where(m, yb, 0.0)
        stats[0:1, :] += jnp.sum(yam, axis=0, keepdims=True)
        stats[1:2, :] += jnp.sum(ybm, axis=0, keepdims=True)
        stats[2:3, :] += jnp.sum(yam * yam, axis=0, keepdims=True)
        stats[3:4, :] += jnp.sum(ybm * ybm, axis=0, keepdims=True)

    @pl.when(ph == 1)
    def _():
        inv_n = jnp.float32(1.0 / N)
        ma = stats[0:1, :] * inv_n
        mb = stats[1:2, :] * inv_n
        va = stats[2:3, :] * inv_n - ma * ma
        vb = stats[3:4, :] * inv_n - mb * mb
        sa = lax.rsqrt(va + EPS) * g_ref[0:1, :]
        sb = lax.rsqrt(vb + EPS) * g_ref[1:2, :]
        za = jnp.maximum((ya - ma) * sa + b_ref[0:1, :], 0.0)
        zb = jnp.maximum((yb - mb) * sb + b_ref[1:2, :], 0.0)
        t = (jnp.dot(za, w2_ref[:HID, :], preferred_element_type=jnp.float32)
             + jnp.dot(zb, w2_ref[HID:, :],
                       preferred_element_type=jnp.float32))
        o_ref[...] = t * dinv


def _mid_call(agg1, dinv, g1, b1r, W2):
    return pl.pallas_call(
        _mid_body,
        grid=(2, NB),
        in_specs=[
            pl.BlockSpec((1, BM, HID), lambda p, i: (0, i, 0)),
            pl.BlockSpec((1, BM, HID), lambda p, i: (1, i, 0)),
            pl.BlockSpec((BM, 1), lambda p, i: (i, 0)),
            pl.BlockSpec((2, HID), lambda p, i: (0, 0)),
            pl.BlockSpec((2, HID), lambda p, i: (0, 0)),
            pl.BlockSpec((H2, HID), lambda p, i: (0, 0)),
        ],
        out_specs=pl.BlockSpec((BM, HID), lambda p, i: (p * i, 0)),
        out_shape=jax.ShapeDtypeStruct((NP, HID), jnp.float32),
        scratch_shapes=[pltpu.VMEM((4, HID), jnp.float32)],
    )(agg1, agg1, dinv, g1, b1r, W2)


def _final_body(e_ref, dinv_ref, g_ref, b_ref, batch_ref,
                wn_ref, bn_ref, wf_ref, bf_ref, out_ref, stats, pooled):
    ph = pl.program_id(0)
    blk = pl.program_id(1)
    dinv = dinv_ref[...]
    y = e_ref[0] * dinv
    rid = blk * BM + lax.broadcasted_iota(jnp.int32, (BM, HID), 0)
    m = rid < N

    @pl.when((ph == 0) & (blk == 0))
    def _():
        stats[...] = jnp.zeros_like(stats)
        pooled[...] = jnp.zeros_like(pooled)

    @pl.when(ph == 0)
    def _():
        ym = jnp.where(m, y, 0.0)
        stats[0:1, :] += jnp.sum(ym, axis=0, keepdims=True)
        stats[1:2, :] += jnp.sum(ym * ym, axis=0, keepdims=True)

    @pl.when(ph == 1)
    def _():
        inv_n = jnp.float32(1.0 / N)
        mean = stats[0:1, :] * inv_n
        var = stats[1:2, :] * inv_n - mean * mean
        z = jnp.maximum((y - mean) * lax.rsqrt(var + EPS) * g_ref[...]
                        + b_ref[...], 0.0)
        zm = jnp.where(m, z, 0.0)
        oh = (batch_ref[...] ==
              lax.broadcasted_iota(jnp.int32, (BM, G), 1)).astype(jnp.float32)
        pooled[...] += lax.dot_general(
            oh, zm, (((0,), (0,)), ((), ())),
            preferred_element_type=jnp.float32)

    @pl.when((ph == 1) & (blk == NB - 1))
    def _():
        t = jnp.dot(pooled[...], wn_ref[...],
                    preferred_element_type=jnp.float32) + bn_ref[...]
        lg = jnp.dot(t, wf_ref[...],
                     preferred_element_type=jnp.float32) + bf_ref[...]
        mx = jnp.max(lg, axis=1, keepdims=True)
        lse = jnp.log(jnp.sum(jnp.exp(lg - mx), axis=1, keepdims=True))
        out_ref[...] = lg - mx - lse


def _final_call(agg2, dinv, g2, b2r, batchp, Wn, bnr, Wf, bfr):
    return pl.pallas_call(
        _final_body,
        grid=(2, NB),
        in_specs=[
            pl.BlockSpec((1, BM, HID), lambda p, i: (0, i, 0)),
            pl.BlockSpec((BM, 1), lambda p, i: (i, 0)),
            pl.BlockSpec((1, HID), lambda p, i: (0, 0)),
            pl.BlockSpec((1, HID), lambda p, i: (0, 0)),
            pl.BlockSpec((BM, 1), lambda p, i: (i, 0)),
            pl.BlockSpec((HID, HID), lambda p, i: (0, 0)),
            pl.BlockSpec((1, HID), lambda p, i: (0, 0)),
            pl.BlockSpec((HID, NCLS), lambda p, i: (0, 0)),
            pl.BlockSpec((1, NCLS), lambda p, i: (0, 0)),
        ],
        out_specs=pl.BlockSpec((G, NCLS), lambda p, i: (0, 0)),
        out_shape=jax.ShapeDtypeStruct((G, NCLS), jnp.float32),
        scratch_shapes=[
            pltpu.VMEM((2, HID), jnp.float32),
            pltpu.VMEM((G, HID), jnp.float32),
        ],
    )(agg2, dinv, g2, b2r, batchp, Wn, bnr, Wf, bfr)


def kernel(x, edge_index, batch, W1, b1, gamma1, beta1, W2, b2, gamma2,
           beta2, Wn, bn, Wf, bf):
    padv = N + (jnp.arange(E_PAD - E, dtype=jnp.int32) % (NP - N))
    srcr = jnp.concatenate([edge_index[0], padv]).reshape(E_PAD // CH, CH)
    dstr = jnp.concatenate([edge_index[1], padv]).reshape(E_PAD // CH, CH)
    xp = jnp.pad(x, ((0, NP - N), (0, 0)))
    batchp = jnp.pad(batch, (0, NP - N), constant_values=G).reshape(NP, 1)

    sre = srcr * 2
    sro = sre + 1
    iide = (2 * jnp.arange(NP, dtype=jnp.int32)).reshape(NP // CH, CH)
    iido = iide + 1
    iids = [iide, iido]

    deg = _deg_call(dstr)
    h1a, h1b, dinv = _lin1_call(xp, W1, deg)
    agg1 = _agg_call(2, [h1a.reshape(2 * NP, DH), h1b.reshape(2 * NP, DH)],
                     iids, [sre, sro, sre, sro], dstr)
    h2 = _mid_call(agg1, dinv, gamma1.reshape(2, HID),
                   beta1.reshape(2, HID), W2)
    h2g = h2.reshape(2 * NP, DH)
    agg2 = _agg_call(1, [h2g, h2g], iids, [sre, sro], dstr)
    return _final_call(agg2, dinv, gamma2.reshape(1, HID),
                       beta2.reshape(1, HID), batchp, Wn,
                       bn.reshape(1, HID), Wf, bf.reshape(1, NCLS))


# speedup vs baseline: 23.3898x; 1.0121x over previous
import functools

import jax
import jax.numpy as jnp
from jax import lax
from jax.experimental import pallas as pl
from jax.experimental.pallas import tpu as pltpu
from jax.experimental.pallas import tpu_sc as plsc

N = 10000
NP = 10240
G = 128
NCLS = 40
DIN = 128
HID = 128
H2 = 256
EPS = 1e-5

CH = 128
TILES = 16
STRIPE = NP // TILES
E = 320000
T_CH = 160
E_PAD = TILES * T_CH * CH
DEG_CH = T_CH // 2

BM = 512
NB = NP // BM

_mesh = plsc.VectorSubcoreMesh(core_axis_name="c", subcore_axis_name="s")


def _fill2d(ref, rows, cols, value):
    vec = jnp.full((16,), value, jnp.float32)

    def body(r, _):
        for c in range(cols // 16):
            ref[r, pl.ds(c * 16, 16)] = vec
        return 0

    lax.fori_loop(0, rows, body, 0)


def _deg_body(dstr, deg_out, idx_v, ones_v, deg_sp):
    cid = lax.axis_index("c")
    wid = lax.axis_index("s")
    row0 = wid * STRIPE
    _fill2d(ones_v, CH, 16, 0.0)
    for k in range(STRIPE // CH):
        pltpu.sync_copy(ones_v, deg_sp.at[pl.ds(row0 + k * CH, CH)])
    _fill2d(ones_v, CH, 16, 1.0)
    base = (cid * TILES + wid) * DEG_CH
    pltpu.sync_copy(dstr.at[pl.ds(base, DEG_CH)], idx_v)
    plsc.subcore_barrier()

    def step(j, _):
        pltpu.sync_copy(ones_v, deg_sp.at[idx_v.at[j]], add=True)
        return 0

    lax.fori_loop(0, DEG_CH, step, 0)
    plsc.subcore_barrier()
    pltpu.sync_copy(deg_sp.at[pl.ds(row0, STRIPE)],
                    deg_out.at[cid, pl.ds(row0, STRIPE)])


def _deg_call(dstr):
    return pl.kernel(
        _deg_body,
        out_type=jax.ShapeDtypeStruct((2, NP, 16), jnp.float32),
        mesh=_mesh,
        scratch_types=[
            pltpu.VMEM((DEG_CH, CH), jnp.int32),
            pltpu.VMEM((CH, 16), jnp.float32),
            pltpu.VMEM_SHARED((NP, 16), jnp.float32),
        ],
        compiler_params=pltpu.CompilerParams(use_tc_tiling_on_sc=False),
    )(dstr)


DH = 64
NBUF = 5
NGRP = T_CH // NBUF


def _agg_body(npass, *args):
    hg = args[:2]
    iids = args[2:4]
    srs = args[4:4 + 2 * npass]
    (dstr, out, sidx, didx, rows, acc_sp) = args[4 + 2 * npass:10 + 2 * npass]
    sems = args[10 + 2 * npass:]
    gsem = sems[:NBUF]
    ssem = sems[NBUF:]
    cid = lax.axis_index("c")
    wid = lax.axis_index("s")
    row0 = wid * STRIPE
    pltpu.sync_copy(dstr.at[pl.ds(wid * T_CH, T_CH)], didx)

    def run(h, sr):
        pltpu.sync_copy(sr.at[pl.ds(wid * T_CH, T_CH)], sidx)
        for b in range(NBUF):
            pltpu.async_copy(h.at[sidx.at[b]], rows.at[b], gsem[b])

        def step(i, _):
            for b in range(NBUF):
                pltpu.make_async_copy(
                    h.at[sidx.at[NBUF * i + b]], rows.at[b], gsem[b]).wait()
                pltpu.async_copy(rows.at[b],
                                 acc_sp.at[didx.at[NBUF * i + b]],
                                 ssem[b], add=True)
            for b in range(NBUF):
                jn = jnp.minimum(NBUF * (i + 1) + b, T_CH - 1)
                pltpu.make_async_copy(
                    rows.at[b], acc_sp.at[didx.at[NBUF * i + b]],
                    ssem[b]).wait()
                pltpu.async_copy(h.at[sidx.at[jn]], rows.at[b], gsem[b])
            return 0

        lax.fori_loop(0, NGRP, step, 0)
        for b in range(NBUF):
            pltpu.make_async_copy(
                h.at[sidx.at[T_CH - 1]], rows.at[b], gsem[b]).wait()

    def init_acc(h, iid):
        nk = STRIPE // CH
        pltpu.sync_copy(iid.at[pl.ds(wid * nk, nk)], sidx.at[pl.ds(0, nk)])
        for k in range(nk):
            pltpu.async_copy(h.at[sidx.at[k]], rows.at[0], gsem[0]).wait()
            pltpu.sync_copy(rows.at[0],
                            acc_sp.at[pl.ds(row0 + k * CH, CH)])

    for p in range(npass):
        for c in range(2):
            g = c * npass + p
            pl.when(cid == c)(functools.partial(init_acc, hg[c],
                                                iids[g % 2]))
        plsc.subcore_barrier()
        for c in range(2):
            pl.when(cid == c)(functools.partial(run, hg[c],
                                                srs[c * npass + p]))
        plsc.subcore_barrier()
        g = cid * npass + p
        pltpu.sync_copy(
            acc_sp.at[pl.ds(row0, STRIPE)],
            out.at[g // 2, pl.ds(row0, STRIPE), pl.ds((g % 2) * DH, DH)])
        plsc.subcore_barrier()


def _agg_call(npass, hg, iids, srs, dstr):
    return pl.kernel(
        functools.partial(_agg_body, npass),
        out_type=jax.ShapeDtypeStruct((npass, NP, 2 * DH), jnp.float32),
        mesh=_mesh,
        scratch_types=[
            pltpu.VMEM((T_CH, CH), jnp.int32),
            pltpu.VMEM((T_CH, CH), jnp.int32),
            pltpu.VMEM((NBUF, CH, DH), jnp.float32),
            pltpu.VMEM_SHARED((NP, DH), jnp.float32),
        ] + [pltpu.SemaphoreType.DMA] * (2 * NBUF),
        compiler_params=pltpu.CompilerParams(use_tc_tiling_on_sc=False),
    )(*hg, *iids, *srs, dstr)


def _lin1_body(x_ref, w_ref, dega_ref, degb_ref, ha_ref, hb_ref, dinv_ref):
    deg = dega_ref[0][:, 0:1] + degb_ref[0][:, 0:1] + 1.0
    dinv = lax.rsqrt(deg)
    h = jnp.dot(x_ref[...], w_ref[...], preferred_element_type=jnp.float32)
    hs = h * dinv
    ha_ref[...] = hs[:, :HID]
    hb_ref[...] = hs[:, HID:]
    dinv_ref[...] = dinv


def _lin1_call(xp, W1, deg):
    hspec = pl.BlockSpec((BM, HID), lambda i: (i, 0))
    hshape = jax.ShapeDtypeStruct((NP, HID), jnp.float32)
    return pl.pallas_call(
        _lin1_body,
        grid=(NB,),
        in_specs=[
            pl.BlockSpec((BM, DIN), lambda i: (i, 0)),
            pl.BlockSpec((DIN, H2), lambda i: (0, 0)),
            pl.BlockSpec((1, BM, 16), lambda i: (0, i, 0)),
            pl.BlockSpec((1, BM, 16), lambda i: (1, i, 0)),
        ],
        out_specs=[hspec, hspec,
                   pl.BlockSpec((BM, 1), lambda i: (i, 0))],
        out_shape=[hshape, hshape,
                   jax.ShapeDtypeStruct((NP, 1), jnp.float32)],
    )(xp, W1, deg, deg)


def _mid_body(atop_ref, abot_ref, dinv_ref, g_ref, b_ref,
              w2_ref, o_ref, stats):
    ph = pl.program_id(0)
    blk = pl.program_id(1)
    dinv = dinv_ref[...]
    ya = atop_ref[0] * dinv
    yb = abot_ref[0] * dinv

    @pl.when((ph == 0) & (blk == 0))
    def _():
        stats[...] = jnp.zeros_like(stats)

    @pl.when(ph == 0)
    def _():
        rid = blk * BM + lax.broadcasted_iota(jnp.int32, (BM, HID), 0)
        m = rid < N
        yam = jnp.where(m, ya, 0.0)
        ybm = jnp.where(m, yb, 0.0)
        stats[0:1, :] += jnp.sum(yam, axis=0, keepdims=True)
        stats[1:2, :] += jnp.sum(ybm, axis=0, keepdims=True)
        stats[2:3, :] += jnp.sum(yam * yam, axis=0, keepdims=True)
        stats[3:4, :] += jnp.sum(ybm * ybm, axis=0, keepdims=True)

    @pl.when(ph == 1)
    def _():
        inv_n = jnp.float32(1.0 / N)
        ma = stats[0:1, :] * inv_n
        mb = stats[1:2, :] * inv_n
        va = stats[2:3, :] * inv_n - ma * ma
        vb = stats[3:4, :] * inv_n - mb * mb
        sa = lax.rsqrt(va + EPS) * g_ref[0:1, :]
        sb = lax.rsqrt(vb + EPS) * g_ref[1:2, :]
        za = jnp.maximum((ya - ma) * sa + b_ref[0:1, :], 0.0)
        zb = jnp.maximum((yb - mb) * sb + b_ref[1:2, :], 0.0)
        t = (jnp.dot(za, w2_ref[:HID, :], preferred_element_type=jnp.float32)
             + jnp.dot(zb, w2_ref[HID:, :],
                       preferred_element_type=jnp.float32))
        o_ref[...] = t * dinv


def _mid_call(agg1, dinv, g1, b1r, W2):
    return pl.pallas_call(
        _mid_body,
        grid=(2, NB),
        in_specs=[
            pl.BlockSpec((1, BM, HID), lambda p, i: (0, i, 0)),
            pl.BlockSpec((1, BM, HID), lambda p, i: (1, i, 0)),
            pl.BlockSpec((BM, 1), lambda p, i: (i, 0)),
            pl.BlockSpec((2, HID), lambda p, i: (0, 0)),
            pl.BlockSpec((2, HID), lambda p, i: (0, 0)),
            pl.BlockSpec((H2, HID), lambda p, i: (0, 0)),
        ],
        out_specs=pl.BlockSpec((BM, HID), lambda p, i: (p * i, 0)),
        out_shape=jax.ShapeDtypeStruct((NP, HID), jnp.float32),
        scratch_shapes=[pltpu.VMEM((4, HID), jnp.float32)],
    )(agg1, agg1, dinv, g1, b1r, W2)


def _final_body(e_ref, dinv_ref, g_ref, b_ref, batch_ref,
                wn_ref, bn_ref, wf_ref, bf_ref, out_ref, stats, pooled):
    ph = pl.program_id(0)
    blk = pl.program_id(1)
    dinv = dinv_ref[...]
    y = e_ref[0] * dinv
    rid = blk * BM + lax.broadcasted_iota(jnp.int32, (BM, HID), 0)
    m = rid < N

    @pl.when((ph == 0) & (blk == 0))
    def _():
        stats[...] = jnp.zeros_like(stats)
        pooled[...] = jnp.zeros_like(pooled)

    @pl.when(ph == 0)
    def _():
        ym = jnp.where(m, y, 0.0)
        stats[0:1, :] += jnp.sum(ym, axis=0, keepdims=True)
        stats[1:2, :] += jnp.sum(ym * ym, axis=0, keepdims=True)

    @pl.when(ph == 1)
    def _():
        inv_n = jnp.float32(1.0 / N)
        mean = stats[0:1, :] * inv_n
        var = stats[1:2, :] * inv_n - mean * mean
        z = jnp.maximum((y - mean) * lax.rsqrt(var + EPS) * g_ref[...]
                        + b_ref[...], 0.0)
        zm = jnp.where(m, z, 0.0)
        oh = (batch_ref[...] ==
              lax.broadcasted_iota(jnp.int32, (BM, G), 1)).astype(jnp.float32)
        pooled[...] += lax.dot_general(
            oh, zm, (((0,), (0,)), ((), ())),
            preferred_element_type=jnp.float32)

    @pl.when((ph == 1) & (blk == NB - 1))
    def _():
        t = jnp.dot(pooled[...], wn_ref[...],
                    preferred_element_type=jnp.float32) + bn_ref[...]
        lg = jnp.dot(t, wf_ref[...],
                     preferred_element_type=jnp.float32) + bf_ref[...]
        mx = jnp.max(lg, axis=1, keepdims=True)
        lse = jnp.log(jnp.sum(jnp.exp(lg - mx), axis=1, keepdims=True))
        out_ref[...] = lg - mx - lse


def _final_call(agg2, dinv, g2, b2r, batchp, Wn, bnr, Wf, bfr):
    return pl.pallas_call(
        _final_body,
        grid=(2, NB),
        in_specs=[
            pl.BlockSpec((1, BM, HID), lambda p, i: (0, i, 0)),
            pl.BlockSpec((BM, 1), lambda p, i: (i, 0)),
            pl.BlockSpec((1, HID), lambda p, i: (0, 0)),
            pl.BlockSpec((1, HID), lambda p, i: (0, 0)),
            pl.BlockSpec((BM, 1), lambda p, i: (i, 0)),
            pl.BlockSpec((HID, HID), lambda p, i: (0, 0)),
            pl.BlockSpec((1, HID), lambda p, i: (0, 0)),
            pl.BlockSpec((HID, NCLS), lambda p, i: (0, 0)),
            pl.BlockSpec((1, NCLS), lambda p, i: (0, 0)),
        ],
        out_specs=pl.BlockSpec((G, NCLS), lambda p, i: (0, 0)),
        out_shape=jax.ShapeDtypeStruct((G, NCLS), jnp.float32),
        scratch_shapes=[
            pltpu.VMEM((2, HID), jnp.float32),
            pltpu.VMEM((G, HID), jnp.float32),
        ],
    )(agg2, dinv, g2, b2r, batchp, Wn, bnr, Wf, bfr)


def kernel(x, edge_index, batch, W1, b1, gamma1, beta1, W2, b2, gamma2,
           beta2, Wn, bn, Wf, bf):
    padv = N + (jnp.arange(E_PAD - E, dtype=jnp.int32) % (NP - N))
    srcr = jnp.concatenate([edge_index[0], padv]).reshape(E_PAD // CH, CH)
    dstr = jnp.concatenate([edge_index[1], padv]).reshape(E_PAD // CH, CH)
    xp = jnp.pad(x, ((0, NP - N), (0, 0)))
    batchp = jnp.pad(batch, (0, NP - N), constant_values=G).reshape(NP, 1)

    sre = srcr * 2
    sro = sre + 1
    iide = (2 * jnp.arange(NP, dtype=jnp.int32)).reshape(NP // CH, CH)
    iido = iide + 1
    iids = [iide, iido]

    deg = _deg_call(dstr)
    h1a, h1b, dinv = _lin1_call(xp, W1, deg)
    agg1 = _agg_call(2, [h1a.reshape(2 * NP, DH), h1b.reshape(2 * NP, DH)],
                     iids, [sre, sro, sre, sro], dstr)
    h2 = _mid_call(agg1, dinv, gamma1.reshape(2, HID),
                   beta1.reshape(2, HID), W2)
    h2g = h2.reshape(2 * NP, DH)
    agg2 = _agg_call(1, [h2g, h2g], iids, [sre, sro], dstr)
    return _final_call(agg2, dinv, gamma2.reshape(1, HID),
                       beta2.reshape(1, HID), batchp, Wn,
                       bn.reshape(1, HID), Wf, bf.reshape(1, NCLS))


# speedup vs baseline: 23.3952x; 1.0002x over previous
import functools

import jax
import jax.numpy as jnp
from jax import lax
from jax.experimental import pallas as pl
from jax.experimental.pallas import tpu as pltpu
from jax.experimental.pallas import tpu_sc as plsc

N = 10000
NP = 10240
G = 128
NCLS = 40
DIN = 128
HID = 128
H2 = 256
EPS = 1e-5

CH = 128
TILES = 16
STRIPE = NP // TILES
E = 320000
T_CH = 160
E_PAD = TILES * T_CH * CH
DEG_CH = T_CH // 2

BM = 512
NB = NP // BM

_mesh = plsc.VectorSubcoreMesh(core_axis_name="c", subcore_axis_name="s")

NFOLD = BM // CH


def _fold_masks():
    rk = lax.broadcasted_iota(jnp.int32, (BM, NFOLD), 0) // CH
    ik = (rk == lax.broadcasted_iota(jnp.int32, (BM, NFOLD), 1))
    rl = lax.broadcasted_iota(jnp.int32, (BM, CH), 0) % CH
    il = (rl == lax.broadcasted_iota(jnp.int32, (BM, CH), 1))
    return ik.astype(jnp.float32), il.astype(jnp.float32)


def _pack_col(u):
    ik, il = _fold_masks()
    return lax.dot_general(ik * u, il, (((0,), (0,)), ((), ())),
                           preferred_element_type=jnp.float32)


def _unpack_col(pk):
    ik, il = _fold_masks()
    tmp = lax.dot_general(il, pk, (((1,), (1,)), ((), ())),
                          preferred_element_type=jnp.float32)
    return jnp.sum(tmp * ik, axis=1, keepdims=True)


def _fill2d(ref, rows, cols, value):
    vec = jnp.full((16,), value, jnp.float32)

    def body(r, _):
        for c in range(cols // 16):
            ref[r, pl.ds(c * 16, 16)] = vec
        return 0

    lax.fori_loop(0, rows, body, 0)


def _deg_body(dstr, deg_out, idx_v, ones_v, deg_sp):
    cid = lax.axis_index("c")
    wid = lax.axis_index("s")
    row0 = wid * STRIPE
    _fill2d(ones_v, CH, 16, 0.0)
    for k in range(STRIPE // CH):
        pltpu.sync_copy(ones_v, deg_sp.at[pl.ds(row0 + k * CH, CH)])
    _fill2d(ones_v, CH, 16, 1.0)
    base = (cid * TILES + wid) * DEG_CH
    pltpu.sync_copy(dstr.at[pl.ds(base, DEG_CH)], idx_v)
    plsc.subcore_barrier()

    def step(j, _):
        pltpu.sync_copy(ones_v, deg_sp.at[idx_v.at[j]], add=True)
        return 0

    lax.fori_loop(0, DEG_CH, step, 0)
    plsc.subcore_barrier()
    pltpu.sync_copy(deg_sp.at[pl.ds(row0, STRIPE)],
                    deg_out.at[cid, pl.ds(row0, STRIPE)])


def _deg_call(dstr):
    return pl.kernel(
        _deg_body,
        out_type=jax.ShapeDtypeStruct((2, NP, 16), jnp.float32),
        mesh=_mesh,
        scratch_types=[
            pltpu.VMEM((DEG_CH, CH), jnp.int32),
            pltpu.VMEM((CH, 16), jnp.float32),
            pltpu.VMEM_SHARED((NP, 16), jnp.float32),
        ],
        compiler_params=pltpu.CompilerParams(use_tc_tiling_on_sc=False),
    )(dstr)


DH = 64
NBUF = 5
NGRP = T_CH // NBUF


def _agg_body(npass, *args):
    hg = args[:2]
    iids = args[2:4]
    srs = args[4:4 + 2 * npass]
    (dstr, out, sidx, didx, rows, acc_sp) = args[4 + 2 * npass:10 + 2 * npass]
    sems = args[10 + 2 * npass:]
    gsem = sems[:NBUF]
    ssem = sems[NBUF:]
    cid = lax.axis_index("c")
    wid = lax.axis_index("s")
    row0 = wid * STRIPE
    pltpu.sync_copy(dstr.at[pl.ds(wid * T_CH, T_CH)], didx)

    def run(h, sr):
        pltpu.sync_copy(sr.at[pl.ds(wid * T_CH, T_CH)], sidx)
        for b in range(NBUF):
            pltpu.async_copy(h.at[sidx.at[b]], rows.at[b], gsem[b])

        def step(i, _):
            for b in range(NBUF):
                pltpu.make_async_copy(
                    h.at[sidx.at[NBUF * i + b]], rows.at[b], gsem[b]).wait()
                pltpu.async_copy(rows.at[b],
                                 acc_sp.at[didx.at[NBUF * i + b]],
                                 ssem[b], add=True)
            for b in range(NBUF):
                jn = jnp.minimum(NBUF * (i + 1) + b, T_CH - 1)
                pltpu.make_async_copy(
                    rows.at[b], acc_sp.at[didx.at[NBUF * i + b]],
                    ssem[b]).wait()
                pltpu.async_copy(h.at[sidx.at[jn]], rows.at[b], gsem[b])
            return 0

        lax.fori_loop(0, NGRP, step, 0)
        for b in range(NBUF):
            pltpu.make_async_copy(
                h.at[sidx.at[T_CH - 1]], rows.at[b], gsem[b]).wait()

    def init_acc(h, iid):
        nk = STRIPE // CH
        pltpu.sync_copy(iid.at[pl.ds(wid * nk, nk)], sidx.at[pl.ds(0, nk)])
        for k in range(nk):
            pltpu.async_copy(h.at[sidx.at[k]], rows.at[0], gsem[0]).wait()
            pltpu.sync_copy(rows.at[0],
                            acc_sp.at[pl.ds(row0 + k * CH, CH)])

    for p in range(npass):
        for c in range(2):
            g = c * npass + p
            pl.when(cid == c)(functools.partial(init_acc, hg[c],
                                                iids[g % 2]))
        plsc.subcore_barrier()
        for c in range(2):
            pl.when(cid == c)(functools.partial(run, hg[c],
                                                srs[c * npass + p]))
        plsc.subcore_barrier()
        g = cid * npass + p
        pltpu.sync_copy(
            acc_sp.at[pl.ds(row0, STRIPE)],
            out.at[g // 2, pl.ds(row0, STRIPE), pl.ds((g % 2) * DH, DH)])
        plsc.subcore_barrier()


def _agg_call(npass, hg, iids, srs, dstr):
    return pl.kernel(
        functools.partial(_agg_body, npass),
        out_type=jax.ShapeDtypeStruct((npass, NP, 2 * DH), jnp.float32),
        mesh=_mesh,
        scratch_types=[
            pltpu.VMEM((T_CH, CH), jnp.int32),
            pltpu.VMEM((T_CH, CH), jnp.int32),
            pltpu.VMEM((NBUF, CH, DH), jnp.float32),
            pltpu.VMEM_SHARED((NP, DH), jnp.float32),
        ] + [pltpu.SemaphoreType.DMA] * (2 * NBUF),
        compiler_params=pltpu.CompilerParams(use_tc_tiling_on_sc=False),
    )(*hg, *iids, *srs, dstr)


def _lin1_body(x_ref, w_ref, dega_ref, degb_ref, ha_ref, hb_ref, dinv_ref):
    deg = dega_ref[0][:, 0:1] + degb_ref[0][:, 0:1] + 1.0
    dinv = lax.rsqrt(deg)
    h = jnp.dot(x_ref[...], w_ref[...], preferred_element_type=jnp.float32)
    hs = h * dinv
    ha_ref[...] = hs[:, :HID]
    hb_ref[...] = hs[:, HID:]
    dinv_ref[...] = _pack_col(dinv)[None]


def _lin1_call(xp, W1, deg):
    hspec = pl.BlockSpec((BM, HID), lambda i: (i, 0))
    hshape = jax.ShapeDtypeStruct((NP, HID), jnp.float32)
    return pl.pallas_call(
        _lin1_body,
        grid=(NB,),
        in_specs=[
            pl.BlockSpec((BM, DIN), lambda i: (i, 0)),
            pl.BlockSpec((DIN, H2), lambda i: (0, 0)),
            pl.BlockSpec((1, BM, 16), lambda i: (0, i, 0)),
            pl.BlockSpec((1, BM, 16), lambda i: (1, i, 0)),
        ],
        out_specs=[hspec, hspec,
                   pl.BlockSpec((1, NFOLD, CH), lambda i: (i, 0, 0))],
        out_shape=[hshape, hshape,
                   jax.ShapeDtypeStruct((NB, NFOLD, CH), jnp.float32)],
    )(xp, W1, deg, deg)


def _mid_body(atop_ref, abot_ref, dinv_ref, g_ref, b_ref,
              w2_ref, o_ref, stats):
    ph = pl.program_id(0)
    blk = pl.program_id(1)
    dinv = _unpack_col(dinv_ref[0])
    ya = atop_ref[0] * dinv
    yb = abot_ref[0] * dinv

    @pl.when((ph == 0) & (blk == 0))
    def _():
        stats[...] = jnp.zeros_like(stats)

    @pl.when(ph == 0)
    def _():
        rid = blk * BM + lax.broadcasted_iota(jnp.int32, (BM, HID), 0)
        m = rid < N
        yam = jnp.where(m, ya, 0.0)
        ybm = jnp.where(m, yb, 0.0)
        stats[0:1, :] += jnp.sum(yam, axis=0, keepdims=True)
        stats[1:2, :] += jnp.sum(ybm, axis=0, keepdims=True)
        stats[2:3, :] += jnp.sum(yam * yam, axis=0, keepdims=True)
        stats[3:4, :] += jnp.sum(ybm * ybm, axis=0, keepdims=True)

    @pl.when(ph == 1)
    def _():
        inv_n = jnp.float32(1.0 / N)
        ma = stats[0:1, :] * inv_n
        mb = stats[1:2, :] * inv_n
        va = stats[2:3, :] * inv_n - ma * ma
        vb = stats[3:4, :] * inv_n - mb * mb
        sa = lax.rsqrt(va + EPS) * g_ref[0:1, :]
        sb = lax.rsqrt(vb + EPS) * g_ref[1:2, :]
        za = jnp.maximum((ya - ma) * sa + b_ref[0:1, :], 0.0)
        zb = jnp.maximum((yb - mb) * sb + b_ref[1:2, :], 0.0)
        t = (jnp.dot(za, w2_ref[:HID, :], preferred_element_type=jnp.float32)
             + jnp.dot(zb, w2_ref[HID:, :],
                       preferred_element_type=jnp.float32))
        o_ref[...] = t * dinv


def _mid_call(agg1, dinv, g1, b1r, W2):
    return pl.pallas_call(
        _mid_body,
        grid=(2, NB),
        in_specs=[
            pl.BlockSpec((1, BM, HID), lambda p, i: (0, i, 0)),
            pl.BlockSpec((1, BM, HID), lambda p, i: (1, i, 0)),
            pl.BlockSpec((1, NFOLD, CH), lambda p, i: (i, 0, 0)),
            pl.BlockSpec((2, HID), lambda p, i: (0, 0)),
            pl.BlockSpec((2, HID), lambda p, i: (0, 0)),
            pl.BlockSpec((H2, HID), lambda p, i: (0, 0)),
        ],
        out_specs=pl.BlockSpec((BM, HID), lambda p, i: (p * i, 0)),
        out_shape=jax.ShapeDtypeStruct((NP, HID), jnp.float32),
        scratch_shapes=[pltpu.VMEM((4, HID), jnp.float32)],
    )(agg1, agg1, dinv, g1, b1r, W2)


def _final_body(e_ref, dinv_ref, g_ref, b_ref, batch_ref,
                wn_ref, bn_ref, wf_ref, bf_ref, out_ref, stats, pooled):
    ph = pl.program_id(0)
    blk = pl.program_id(1)
    dinv = _unpack_col(dinv_ref[0])
    y = e_ref[0] * dinv
    rid = blk * BM + lax.broadcasted_iota(jnp.int32, (BM, HID), 0)
    m = rid < N

    @pl.when((ph == 0) & (blk == 0))
    def _():
        stats[...] = jnp.zeros_like(stats)
        pooled[...] = jnp.zeros_like(pooled)

    @pl.when(ph == 0)
    def _():
        ym = jnp.where(m, y, 0.0)
        stats[0:1, :] += jnp.sum(ym, axis=0, keepdims=True)
        stats[1:2, :] += jnp.sum(ym * ym, axis=0, keepdims=True)

    @pl.when(ph == 1)
    def _():
        inv_n = jnp.float32(1.0 / N)
        mean = stats[0:1, :] * inv_n
        var = stats[1:2, :] * inv_n - mean * mean
        z = jnp.maximum((y - mean) * lax.rsqrt(var + EPS) * g_ref[...]
                        + b_ref[...], 0.0)
        zm = jnp.where(m, z, 0.0)
        oh = (batch_ref[...] ==
              lax.broadcasted_iota(jnp.int32, (BM, G), 1)).astype(jnp.float32)
        pooled[...] += lax.dot_general(
            oh, zm, (((0,), (0,)), ((), ())),
            preferred_element_type=jnp.float32)

    @pl.when((ph == 1) & (blk == NB - 1))
    def _():
        t = jnp.dot(pooled[...], wn_ref[...],
                    preferred_element_type=jnp.float32) + bn_ref[...]
        lg = jnp.dot(t, wf_ref[...],
                     preferred_element_type=jnp.float32) + bf_ref[...]
        mx = jnp.max(lg, axis=1, keepdims=True)
        lse = jnp.log(jnp.sum(jnp.exp(lg - mx), axis=1, keepdims=True))
        out_ref[...] = lg - mx - lse


def _final_call(agg2, dinv, g2, b2r, batchp, Wn, bnr, Wf, bfr):
    return pl.pallas_call(
        _final_body,
        grid=(2, NB),
        in_specs=[
            pl.BlockSpec((1, BM, HID), lambda p, i: (0, i, 0)),
            pl.BlockSpec((1, NFOLD, CH), lambda p, i: (i, 0, 0)),
            pl.BlockSpec((1, HID), lambda p, i: (0, 0)),
            pl.BlockSpec((1, HID), lambda p, i: (0, 0)),
            pl.BlockSpec((BM, 1), lambda p, i: (i, 0)),
            pl.BlockSpec((HID, HID), lambda p, i: (0, 0)),
            pl.BlockSpec((1, HID), lambda p, i: (0, 0)),
            pl.BlockSpec((HID, NCLS), lambda p, i: (0, 0)),
            pl.BlockSpec((1, NCLS), lambda p, i: (0, 0)),
        ],
        out_specs=pl.BlockSpec((G, NCLS), lambda p, i: (0, 0)),
        out_shape=jax.ShapeDtypeStruct((G, NCLS), jnp.float32),
        scratch_shapes=[
            pltpu.VMEM((2, HID), jnp.float32),
            pltpu.VMEM((G, HID), jnp.float32),
        ],
    )(agg2, dinv, g2, b2r, batchp, Wn, bnr, Wf, bfr)


def kernel(x, edge_index, batch, W1, b1, gamma1, beta1, W2, b2, gamma2,
           beta2, Wn, bn, Wf, bf):
    padv = N + (jnp.arange(E_PAD - E, dtype=jnp.int32) % (NP - N))
    srcr = jnp.concatenate([edge_index[0], padv]).reshape(E_PAD // CH, CH)
    dstr = jnp.concatenate([edge_index[1], padv]).reshape(E_PAD // CH, CH)
    xp = jnp.pad(x, ((0, NP - N), (0, 0)))
    batchp = jnp.pad(batch, (0, NP - N), constant_values=G).reshape(NP, 1)

    sre = srcr * 2
    sro = sre + 1
    iide = (2 * jnp.arange(NP, dtype=jnp.int32)).reshape(NP // CH, CH)
    iido = iide + 1
    iids = [iide, iido]

    deg = _deg_call(dstr)
    h1a, h1b, dinv = _lin1_call(xp, W1, deg)
    agg1 = _agg_call(2, [h1a.reshape(2 * NP, DH), h1b.reshape(2 * NP, DH)],
                     iids, [sre, sro, sre, sro], dstr)
    h2 = _mid_call(agg1, dinv, gamma1.reshape(2, HID),
                   beta1.reshape(2, HID), W2)
    h2g = h2.reshape(2 * NP, DH)
    agg2 = _agg_call(1, [h2g, h2g], iids, [sre, sro], dstr)
    return _final_call(agg2, dinv, gamma2.reshape(1, HID),
                       beta2.reshape(1, HID), batchp, Wn,
                       bn.reshape(1, HID), Wf, bf.reshape(1, NCLS))
